# baseline (speedup 1.0000x reference)
import functools

import jax
import jax.numpy as jnp
import numpy as np
from jax import lax
from jax.experimental import pallas as pl
from jax.experimental.pallas import tpu as pltpu
from jax.experimental.pallas import tpu_sc as plsc

NUM_CLASSES = 80
ROI = 7
STRIDE = 8
SCORE_THR = 0.05
IOU_THR = 0.5
MAX_PER_IMG = 100
PRE_NMS = 1000
H = 80
W = 80
C = 128
N = 5000
RB = 128
NPAD = 5120
NBLK = NPAD // RB
MAX_RATIO = float(np.abs(np.log(1000.0 / 16.0)))

_INTERP = False
_USE_SC = True


def _head_body(props_ref, ftx_ref, yexp_ref, wcls_ref, bcls_ref, wdx_ref,
               wdy_ref, wdw_ref, wdh_ref, breg_ref, scores_ref,
               bx1_ref, by1_ref, bx2_ref, by2_ref):
    props = props_ref[...]
    x1p = props[:, 0:1]
    y1p = props[:, 1:2]
    x2p = props[:, 2:3]
    y2p = props[:, 3:4]
    scale = 1.0 / STRIDE
    x1 = x1p * scale
    y1 = y1p * scale
    x2 = x2p * scale
    y2 = y2p * scale
    bw = jnp.maximum(x2 - x1, 1e-3) * (1.0 / ROI)
    bh = jnp.maximum(y2 - y1, 1e-3) * (1.0 / ROI)

    def samp_weights(lo, bsz):
        grid = jax.lax.broadcasted_iota(jnp.int32, (RB, W), 1).astype(jnp.float32)
        acc = jnp.zeros((RB, W), jnp.float32)
        for j in range(ROI):
            s = lo + (j + 0.5) * bsz
            f = jnp.floor(s)
            frac = s - f
            i0 = jnp.clip(f, 0.0, W - 1.0)
            i1 = jnp.clip(f + 1.0, 0.0, W - 1.0)
            acc = acc + jnp.where(grid == i0, 1.0 - frac, 0.0) \
                      + jnp.where(grid == i1, frac, 0.0)
        return acc * (1.0 / ROI)

    wx = samp_weights(x1, bw)
    wy = samp_weights(y1, bh)

    t = jax.lax.dot_general(wx, ftx_ref[...], (((1,), (0,)), ((), ())),
                            preferred_element_type=jnp.float32)
    wy_exp = jax.lax.dot_general(wy, yexp_ref[...], (((1,), (0,)), ((), ())),
                                 preferred_element_type=jnp.float32)
    prod = t * wy_exp
    parts = [prod[:, y * C:(y + 1) * C] for y in range(H)]
    while len(parts) > 1:
        nxt = [a + b for a, b in zip(parts[0::2], parts[1::2])]
        if len(parts) % 2:
            nxt[-1] = nxt[-1] + parts[-1]
        parts = nxt
    pooled = parts[0]

    logits = jax.lax.dot_general(pooled, wcls_ref[...], (((1,), (0,)), ((), ())),
                                 preferred_element_type=jnp.float32)
    logits = logits + bcls_ref[...]
    lane = jax.lax.broadcasted_iota(jnp.int32, (RB, 128), 1)
    logits = jnp.where(lane < NUM_CLASSES + 1, logits, -1e30)
    m = jnp.max(logits, axis=1, keepdims=True)
    e = jnp.exp(logits - m)
    ssum = jnp.sum(e, axis=1, keepdims=True)
    scores = e / ssum
    scores = jnp.where(lane < NUM_CLASSES + 1, scores, 0.0)
    gid = pl.program_id(0)
    row = gid * RB + jax.lax.broadcasted_iota(jnp.int32, (RB, 1), 0)
    scores = jnp.where(row < N, scores, 0.0)
    scores_ref[...] = scores

    def reg_head(w_ref, b_ref, std):
        d = jax.lax.dot_general(pooled, w_ref[...], (((1,), (0,)), ((), ())),
                                preferred_element_type=jnp.float32)
        return (d + b_ref[...]) * std

    dx = reg_head(wdx_ref, breg_ref.at[0:1], 0.1)
    dy = reg_head(wdy_ref, breg_ref.at[1:2], 0.1)
    dw = reg_head(wdw_ref, breg_ref.at[2:3], 0.2)
    dh = reg_head(wdh_ref, breg_ref.at[3:4], 0.2)
    dw = jnp.clip(dw, -MAX_RATIO, MAX_RATIO)
    dh = jnp.clip(dh, -MAX_RATIO, MAX_RATIO)

    px = (x1p + x2p) * 0.5
    py = (y1p + y2p) * 0.5
    pw = x2p - x1p
    ph = y2p - y1p
    gx = px + pw * dx
    gy = py + ph * dy
    gw = pw * jnp.exp(dw)
    gh = ph * jnp.exp(dh)
    bx1_ref[...] = gx - gw * 0.5
    by1_ref[...] = gy - gh * 0.5
    bx2_ref[...] = gx + gw * 0.5
    by2_ref[...] = gy + gh * 0.5


_YEXP = np.repeat(np.eye(H, dtype=np.float32), C, axis=1)


def _run_head(props_pad, ftx, wcls_pad, bcls_pad, wdx, wdy, wdw, wdh, breg4):
    full = lambda shape: pl.BlockSpec(shape, lambda i: tuple(0 for _ in shape))
    planar_out = pl.BlockSpec((RB, NUM_CLASSES), lambda i: (i, 0))
    return pl.pallas_call(
        _head_body,
        grid=(NBLK,),
        in_specs=[
            pl.BlockSpec((RB, 4), lambda i: (i, 0)),
            full((W, H * C)),
            full((H, H * C)),
            full((C, 128)),
            full((1, 128)),
            full((C, NUM_CLASSES)),
            full((C, NUM_CLASSES)),
            full((C, NUM_CLASSES)),
            full((C, NUM_CLASSES)),
            full((4, NUM_CLASSES)),
        ],
        out_specs=[
            pl.BlockSpec((RB, 128), lambda i: (i, 0)),
            planar_out, planar_out, planar_out, planar_out,
        ],
        out_shape=[
            jax.ShapeDtypeStruct((NPAD, 128), jnp.float32),
            jax.ShapeDtypeStruct((NPAD, NUM_CLASSES), jnp.float32),
            jax.ShapeDtypeStruct((NPAD, NUM_CLASSES), jnp.float32),
            jax.ShapeDtypeStruct((NPAD, NUM_CLASSES), jnp.float32),
            jax.ShapeDtypeStruct((NPAD, NUM_CLASSES), jnp.float32),
        ],
        compiler_params=pltpu.CompilerParams(
            dimension_semantics=("arbitrary",)),
        interpret=_INTERP,
    )(props_pad, ftx, jnp.asarray(_YEXP), wcls_pad, bcls_pad,
      wdx, wdy, wdw, wdh, breg4)


NT = 32
TPT = NPAD // NT
CCAP = 3072
LCAP = 1024
RPG = 4
EMPTY_IDX = 500000.0
INVAL_IDX = 600000.0


def _tocol(row):
    return jnp.reshape(row, (row.shape[1], 1))


def _merge_into(L_ref, chunk):
    allv = jnp.concatenate([L_ref[...], chunk], axis=1)
    sc_row = allv[0:1, :]
    idx_row = allv[1:2, :]
    sc_col = _tocol(sc_row)
    idx_col = _tocol(idx_row)
    rank_col = jnp.zeros((2 * LCAP, 1), jnp.float32)
    for s in range(4):
        scs = sc_row[:, s * 512:(s + 1) * 512]
        idxs = idx_row[:, s * 512:(s + 1) * 512]
        before = ((scs > sc_col) |
                  ((scs == sc_col) & (idxs < idx_col))).astype(jnp.float32)
        rank_col = rank_col + jnp.sum(before, axis=1, keepdims=True)
    lane = jax.lax.broadcasted_iota(jnp.int32, (1, LCAP), 1).astype(jnp.float32)
    w = (rank_col == lane).astype(jnp.float32)
    L_ref[...] = jax.lax.dot_general(allv, w, (((1,), (0,)), ((), ())),
                                     preferred_element_type=jnp.float32)


def _nms_body(cnt_ref, sc_ref, idx_ref, x1_ref, y1_ref, x2_ref, y2_ref,
              dets_ref, num_ref, L_ref, iou_ref):
    wgrid = pl.program_id(0)
    lane = jax.lax.broadcasted_iota(jnp.int32, (1, LCAP), 1)
    lane_f = lane.astype(jnp.float32)

    @pl.when(wgrid == 0)
    def _init():
        L_ref[...] = jnp.concatenate(
            [jnp.zeros((1, LCAP), jnp.float32),
             EMPTY_IDX + lane_f,
             jnp.zeros((6, LCAP), jnp.float32)], axis=0)

    for k in range(RPG):
        cntw = cnt_ref[wgrid * RPG + k, 0]
        for c in range(CCAP // LCAP):
            @pl.when(cntw > c * LCAP)
            def _do_merge(c=c, k=k, cntw=cntw):
                rem = cntw - c * LCAP
                lm = lane < rem
                raw_sc = sc_ref[k, :, pl.ds(c * LCAP, LCAP)]
                raw_idx = idx_ref[k, :, pl.ds(c * LCAP, LCAP)].astype(
                    jnp.float32)
                csc = jnp.where(lm, raw_sc, -1.0)
                cidx = jnp.where(lm, raw_idx, INVAL_IDX + c * LCAP + lane_f)
                cx1 = jnp.where(lm, x1_ref[k, :, pl.ds(c * LCAP, LCAP)], 0.0)
                cy1 = jnp.where(lm, y1_ref[k, :, pl.ds(c * LCAP, LCAP)], 0.0)
                cx2 = jnp.where(lm, x2_ref[k, :, pl.ds(c * LCAP, LCAP)], 0.0)
                cy2 = jnp.where(lm, y2_ref[k, :, pl.ds(c * LCAP, LCAP)], 0.0)
                chunk = jnp.concatenate(
                    [csc, cidx, cx1, cy1, cx2, cy2,
                     jnp.zeros((2, LCAP), jnp.float32)], axis=0)
                _merge_into(L_ref, chunk)

    @pl.when(wgrid == NT // RPG - 1)
    def _final():
        L = L_ref[...]
        lsc = jnp.where(lane < PRE_NMS, L[0:1, :], 0.0)
        lidx = L[1:2, :]
        x1r = L[2:3, :]
        y1r = L[3:4, :]
        x2r = L[4:5, :]
        y2r = L[5:6, :]
        cls = lidx - jnp.floor(lidx * (1.0 / NUM_CLASSES)) * NUM_CLASSES
        off = cls * 4096.0
        ox1 = x1r + off
        oy1 = y1r + off
        ox2 = x2r + off
        oy2 = y2r + off
        ox1c = _tocol(ox1)
        oy1c = _tocol(oy1)
        ox2c = _tocol(ox2)
        oy2c = _tocol(oy2)
        area_r = jnp.maximum(ox2 - ox1, 0.0) * jnp.maximum(oy2 - oy1, 0.0)
        area_c = jnp.maximum(ox2c - ox1c, 0.0) * jnp.maximum(oy2c - oy1c, 0.0)
        ix1 = jnp.maximum(ox1c, ox1)
        iy1 = jnp.maximum(oy1c, oy1)
        ix2 = jnp.minimum(ox2c, ox2)
        iy2 = jnp.minimum(oy2c, oy2)
        inter = jnp.maximum(ix2 - ix1, 0.0) * jnp.maximum(iy2 - iy1, 0.0)
        iou_ref[...] = inter / (area_c + area_r - inter + 1e-6)

        npos = jnp.sum(jnp.where(lsc > 0.0, 1, 0))

        def body(i, keep):
            row = iou_ref[pl.ds(i, 1), :]
            ki = jnp.sum(jnp.where(lane == i, keep, 0.0))
            sup = (row > IOU_THR) & (lane > i) & (ki > 0.0)
            return jnp.where(sup, 0.0, keep)

        keep0 = jnp.where(lsc > 0.0, 1.0, 0.0)
        kept = jax.lax.fori_loop(0, npos, body, keep0)

        sub2d = jax.lax.broadcasted_iota(jnp.int32, (LCAP, LCAP), 0)
        lane2d = jax.lax.broadcasted_iota(jnp.int32, (LCAP, LCAP), 1)
        m3 = jnp.where(lane2d < sub2d, kept, 0.0)
        pr_col = jnp.sum(m3, axis=1, keepdims=True)
        lane128 = jax.lax.broadcasted_iota(
            jnp.int32, (1, 128), 1).astype(jnp.float32)
        wd = (pr_col == lane128).astype(jnp.float32)
        dmat = jnp.concatenate(
            [lsc, cls, x1r, y1r, x2r, y2r,
             jnp.zeros((2, LCAP), jnp.float32)], axis=0) * kept
        dets_ref[...] = jax.lax.dot_general(
            dmat, wd, (((1,), (0,)), ((), ())),
            preferred_element_type=jnp.float32)
        nk = jnp.sum(kept).astype(jnp.int32)
        num_ref[0, 0] = jnp.minimum(nk, MAX_PER_IMG)


def _run_nms(cnt2d, csc, cidx, cx1, cy1, cx2, cy2):
    cand_spec = pl.BlockSpec((RPG, 1, CCAP), lambda i: (i, 0, 0))
    return pl.pallas_call(
        _nms_body,
        grid=(NT // RPG,),
        in_specs=[
            pl.BlockSpec(memory_space=pltpu.SMEM),
            cand_spec, cand_spec, cand_spec, cand_spec, cand_spec, cand_spec,
        ],
        out_specs=[
            pl.BlockSpec((8, 128), lambda i: (0, 0)),
            pl.BlockSpec(memory_space=pltpu.SMEM),
        ],
        out_shape=[
            jax.ShapeDtypeStruct((8, 128), jnp.float32),
            jax.ShapeDtypeStruct((1, 1), jnp.int32),
        ],
        scratch_shapes=[
            pltpu.VMEM((8, LCAP), jnp.float32),
            pltpu.VMEM((LCAP, LCAP), jnp.float32),
        ],
        compiler_params=pltpu.CompilerParams(
            dimension_semantics=("arbitrary",)),
        interpret=_INTERP,
    )(cnt2d, csc, cidx, cx1, cy1, cx2, cy2)


def _sc_compact_body(scores_hbm, bx1_hbm, by1_hbm, bx2_hbm, by2_hbm,
                     cnt_out, sc_out, idx_out, x1_out, y1_out, x2_out, y2_out,
                     sc_v, bx1_v, by1_v, bx2_v, by2_v,
                     csc_v, cidx_v, cx1_v, cy1_v, cx2_v, cy2_v,
                     cnt_v):
    nc = 2
    wid = lax.axis_index("s") * nc + lax.axis_index("c")
    base = wid * TPT
    pltpu.sync_copy(scores_hbm.at[pl.ds(base, TPT)], sc_v)
    pltpu.sync_copy(bx1_hbm.at[pl.ds(base, TPT)], bx1_v)
    pltpu.sync_copy(by1_hbm.at[pl.ds(base, TPT)], by1_v)
    pltpu.sync_copy(bx2_hbm.at[pl.ds(base, TPT)], bx2_v)
    pltpu.sync_copy(by2_hbm.at[pl.ds(base, TPT)], by2_v)
    lane16 = lax.broadcasted_iota(jnp.int32, (16,), 0)

    def _prefix_inclusive(x):
        for d in (1, 2, 4, 8):
            src = jnp.maximum(lane16 - d, 0)
            y = x.at[src].get(mode="promise_in_bounds")
            x = x + jnp.where(lane16 >= d, y, 0.0)
        return x

    def row_body(r, cnt):
        for v in range(NUM_CLASSES // 16):
            s = sc_v[r, pl.ds(v * 16, 16)]
            m = s > SCORE_THR
            mf = jnp.where(m, 1.0, 0.0)
            csum = _prefix_inclusive(mf).astype(jnp.int32)
            pos = (cnt + csum) - 1
            gidx = (base + r) * NUM_CLASSES + v * 16 + lane16
            plsc.store_scatter(csc_v, [pos], s, mask=m)
            plsc.store_scatter(cidx_v, [pos], gidx, mask=m)
            x1v = bx1_v[r, pl.ds(v * 16, 16)]
            y1v = by1_v[r, pl.ds(v * 16, 16)]
            x2v = bx2_v[r, pl.ds(v * 16, 16)]
            y2v = by2_v[r, pl.ds(v * 16, 16)]
            plsc.store_scatter(cx1_v, [pos], x1v, mask=m)
            plsc.store_scatter(cy1_v, [pos], y1v, mask=m)
            plsc.store_scatter(cx2_v, [pos], x2v, mask=m)
            plsc.store_scatter(cy2_v, [pos], y2v, mask=m)
            last = csum.at[jnp.full((16,), 15, jnp.int32)].get(
                mode="promise_in_bounds")
            cnt = cnt + last
        return cnt

    cnt0 = jnp.zeros((16,), jnp.int32)
    cnt = lax.fori_loop(0, TPT, row_body, cnt0)
    cnt_v[...] = cnt
    pltpu.sync_copy(cnt_v, cnt_out.at[pl.ds(wid * 16, 16)])
    pltpu.sync_copy(csc_v, sc_out.at[wid])
    pltpu.sync_copy(cidx_v, idx_out.at[wid])
    pltpu.sync_copy(cx1_v, x1_out.at[wid])
    pltpu.sync_copy(cy1_v, y1_out.at[wid])
    pltpu.sync_copy(cx2_v, x2_out.at[wid])
    pltpu.sync_copy(cy2_v, y2_out.at[wid])


def _run_sc_compact(scores_pad, bx1, by1, bx2, by2):
    f32 = jnp.float32
    i32 = jnp.int32
    mesh = plsc.VectorSubcoreMesh(core_axis_name="c", subcore_axis_name="s")
    k = functools.partial(
        pl.kernel,
        mesh=mesh,
        out_type=[
            jax.ShapeDtypeStruct((NT * 16,), i32),
            jax.ShapeDtypeStruct((NT, CCAP), f32),
            jax.ShapeDtypeStruct((NT, CCAP), i32),
            jax.ShapeDtypeStruct((NT, CCAP), f32),
            jax.ShapeDtypeStruct((NT, CCAP), f32),
            jax.ShapeDtypeStruct((NT, CCAP), f32),
            jax.ShapeDtypeStruct((NT, CCAP), f32),
        ],
        scratch_types=[
            pltpu.VMEM((TPT, 128), f32),
            pltpu.VMEM((TPT, NUM_CLASSES), f32),
            pltpu.VMEM((TPT, NUM_CLASSES), f32),
            pltpu.VMEM((TPT, NUM_CLASSES), f32),
            pltpu.VMEM((TPT, NUM_CLASSES), f32),
            pltpu.VMEM((CCAP,), f32),
            pltpu.VMEM((CCAP,), i32),
            pltpu.VMEM((CCAP,), f32),
            pltpu.VMEM((CCAP,), f32),
            pltpu.VMEM((CCAP,), f32),
            pltpu.VMEM((CCAP,), f32),
            pltpu.VMEM((16,), i32),
        ],
        compiler_params=pltpu.CompilerParams(needs_layout_passes=False),
    )(_sc_compact_body)
    return k(scores_pad, bx1, by1, bx2, by2)


def _compact_emul_jax(scores_pad, bx1, by1, bx2, by2):
    sc3 = scores_pad[:, :NUM_CLASSES].reshape(NT, TPT * NUM_CLASSES)
    m = sc3 > SCORE_THR
    cnt = jnp.sum(m.astype(jnp.int32), axis=1)
    order = jnp.argsort(~m, axis=1, stable=True)[:, :CCAP]
    csc = jnp.take_along_axis(sc3, order, axis=1)
    base = (jnp.arange(NT, dtype=jnp.int32) * TPT * NUM_CLASSES)[:, None]
    cidx = base + order.astype(jnp.int32)
    outs = [jnp.take_along_axis(b.reshape(NT, TPT * NUM_CLASSES), order, axis=1)
            for b in (bx1, by1, bx2, by2)]
    return (cnt, csc, cidx, *outs)


def kernel(feat, proposals, W_cls, b_cls, W_reg, b_reg):
    ftx = jnp.transpose(feat[0], (2, 1, 0)).reshape(W, H * C)
    props_pad = jnp.pad(proposals[0], ((0, NPAD - N), (0, 0)))
    wcls_pad = jnp.pad(W_cls, ((0, 0), (0, 128 - (NUM_CLASSES + 1))))
    bcls_pad = jnp.pad(b_cls, (0, 128 - (NUM_CLASSES + 1))).reshape(1, 128)
    breg4 = jnp.transpose(b_reg.reshape(NUM_CLASSES, 4))
    wdx = W_reg[:, 0::4]
    wdy = W_reg[:, 1::4]
    wdw = W_reg[:, 2::4]
    wdh = W_reg[:, 3::4]

    scores_pad, bx1, by1, bx2, by2 = _run_head(
        props_pad, ftx, wcls_pad, bcls_pad, wdx, wdy, wdw, wdh, breg4)
    if _USE_SC:
        cntv, csc, cidx, cx1, cy1, cx2, cy2 = _run_sc_compact(
            scores_pad, bx1, by1, bx2, by2)
        cnt2d = cntv.reshape(NT, 16)
    else:
        cnt, csc, cidx, cx1, cy1, cx2, cy2 = _compact_emul_jax(
            scores_pad, bx1, by1, bx2, by2)
        cnt2d = jnp.broadcast_to(cnt[:, None], (NT, 16))
    shp3 = (NT, 1, CCAP)
    dets, num = _run_nms(cnt2d, csc.reshape(shp3), cidx.reshape(shp3),
                         cx1.reshape(shp3), cy1.reshape(shp3),
                         cx2.reshape(shp3), cy2.reshape(shp3))
    det_sc = dets[0, :MAX_PER_IMG]
    det_cls = jnp.where(det_sc > 0.0,
                        dets[1, :MAX_PER_IMG].astype(jnp.int32), -1)
    det_bx = jnp.transpose(dets[2:6, :MAX_PER_IMG])
    num_s = num[0, 0]
    return (num_s[None], det_bx[None], det_sc[None], det_cls[None])


# speedup vs baseline: 2.8752x; 2.8752x over previous
import functools

import jax
import jax.numpy as jnp
import numpy as np
from jax import lax
from jax.experimental import pallas as pl
from jax.experimental.pallas import tpu as pltpu
from jax.experimental.pallas import tpu_sc as plsc

NUM_CLASSES = 80
ROI = 7
STRIDE = 8
SCORE_THR = 0.05
IOU_THR = 0.5
MAX_PER_IMG = 100
PRE_NMS = 1000
H = 80
W = 80
C = 128
N = 5000
RB = 128
NPAD = 5120
NBLK = NPAD // RB
MAX_RATIO = float(np.abs(np.log(1000.0 / 16.0)))

_INTERP = False
_USE_SC = True


def _head_body(props_ref, ftx_ref, yexp_ref, wcls_ref, bcls_ref, wdx_ref,
               wdy_ref, wdw_ref, wdh_ref, breg_ref, scores_ref,
               bx1_ref, by1_ref, bx2_ref, by2_ref):
    props = props_ref[...]
    x1p = props[:, 0:1]
    y1p = props[:, 1:2]
    x2p = props[:, 2:3]
    y2p = props[:, 3:4]
    scale = 1.0 / STRIDE
    x1 = x1p * scale
    y1 = y1p * scale
    x2 = x2p * scale
    y2 = y2p * scale
    bw = jnp.maximum(x2 - x1, 1e-3) * (1.0 / ROI)
    bh = jnp.maximum(y2 - y1, 1e-3) * (1.0 / ROI)

    def samp_weights(lo, bsz):
        grid = jax.lax.broadcasted_iota(jnp.int32, (RB, W), 1).astype(jnp.float32)
        acc = jnp.zeros((RB, W), jnp.float32)
        for j in range(ROI):
            s = lo + (j + 0.5) * bsz
            f = jnp.floor(s)
            frac = s - f
            i0 = jnp.clip(f, 0.0, W - 1.0)
            i1 = jnp.clip(f + 1.0, 0.0, W - 1.0)
            acc = acc + jnp.where(grid == i0, 1.0 - frac, 0.0) \
                      + jnp.where(grid == i1, frac, 0.0)
        return acc * (1.0 / ROI)

    wx = samp_weights(x1, bw)
    wy = samp_weights(y1, bh)

    t = jax.lax.dot_general(wx, ftx_ref[...], (((1,), (0,)), ((), ())),
                            preferred_element_type=jnp.float32)
    wy_exp = jax.lax.dot_general(wy, yexp_ref[...], (((1,), (0,)), ((), ())),
                                 preferred_element_type=jnp.float32)
    prod = t * wy_exp
    parts = [prod[:, y * C:(y + 1) * C] for y in range(H)]
    while len(parts) > 1:
        nxt = [a + b for a, b in zip(parts[0::2], parts[1::2])]
        if len(parts) % 2:
            nxt[-1] = nxt[-1] + parts[-1]
        parts = nxt
    pooled = parts[0]

    logits = jax.lax.dot_general(pooled, wcls_ref[...], (((1,), (0,)), ((), ())),
                                 preferred_element_type=jnp.float32)
    logits = logits + bcls_ref[...]
    lane = jax.lax.broadcasted_iota(jnp.int32, (RB, 128), 1)
    logits = jnp.where(lane < NUM_CLASSES + 1, logits, -1e30)
    m = jnp.max(logits, axis=1, keepdims=True)
    e = jnp.exp(logits - m)
    ssum = jnp.sum(e, axis=1, keepdims=True)
    scores = e / ssum
    scores = jnp.where(lane < NUM_CLASSES + 1, scores, 0.0)
    gid = pl.program_id(0)
    row = gid * RB + jax.lax.broadcasted_iota(jnp.int32, (RB, 1), 0)
    scores = jnp.where(row < N, scores, 0.0)
    scores_ref[...] = scores

    def reg_head(w_ref, b_ref, std):
        d = jax.lax.dot_general(pooled, w_ref[...], (((1,), (0,)), ((), ())),
                                preferred_element_type=jnp.float32)
        return (d + b_ref[...]) * std

    dx = reg_head(wdx_ref, breg_ref.at[0:1], 0.1)
    dy = reg_head(wdy_ref, breg_ref.at[1:2], 0.1)
    dw = reg_head(wdw_ref, breg_ref.at[2:3], 0.2)
    dh = reg_head(wdh_ref, breg_ref.at[3:4], 0.2)
    dw = jnp.clip(dw, -MAX_RATIO, MAX_RATIO)
    dh = jnp.clip(dh, -MAX_RATIO, MAX_RATIO)

    px = (x1p + x2p) * 0.5
    py = (y1p + y2p) * 0.5
    pw = x2p - x1p
    ph = y2p - y1p
    gx = px + pw * dx
    gy = py + ph * dy
    gw = pw * jnp.exp(dw)
    gh = ph * jnp.exp(dh)
    bx1_ref[...] = gx - gw * 0.5
    by1_ref[...] = gy - gh * 0.5
    bx2_ref[...] = gx + gw * 0.5
    by2_ref[...] = gy + gh * 0.5


_YEXP = np.repeat(np.eye(H, dtype=np.float32), C, axis=1)


def _run_head(props_pad, ftx, wcls_pad, bcls_pad, wdx, wdy, wdw, wdh, breg4):
    full = lambda shape: pl.BlockSpec(shape, lambda i: tuple(0 for _ in shape))
    planar_out = pl.BlockSpec((RB, NUM_CLASSES), lambda i: (i, 0))
    return pl.pallas_call(
        _head_body,
        grid=(NBLK,),
        in_specs=[
            pl.BlockSpec((RB, 4), lambda i: (i, 0)),
            full((W, H * C)),
            full((H, H * C)),
            full((C, 128)),
            full((1, 128)),
            full((C, NUM_CLASSES)),
            full((C, NUM_CLASSES)),
            full((C, NUM_CLASSES)),
            full((C, NUM_CLASSES)),
            full((4, NUM_CLASSES)),
        ],
        out_specs=[
            pl.BlockSpec((RB, 128), lambda i: (i, 0)),
            planar_out, planar_out, planar_out, planar_out,
        ],
        out_shape=[
            jax.ShapeDtypeStruct((NPAD, 128), jnp.float32),
            jax.ShapeDtypeStruct((NPAD, NUM_CLASSES), jnp.float32),
            jax.ShapeDtypeStruct((NPAD, NUM_CLASSES), jnp.float32),
            jax.ShapeDtypeStruct((NPAD, NUM_CLASSES), jnp.float32),
            jax.ShapeDtypeStruct((NPAD, NUM_CLASSES), jnp.float32),
        ],
        compiler_params=pltpu.CompilerParams(
            dimension_semantics=("arbitrary",)),
        interpret=_INTERP,
    )(props_pad, ftx, jnp.asarray(_YEXP), wcls_pad, bcls_pad,
      wdx, wdy, wdw, wdh, breg4)


NT = 32
TPT = NPAD // NT
CCAP = 3072
LCAP = 1024
RPG = 1
EMPTY_IDX = 500000.0
INVAL_IDX = 600000.0


def _tocol(row):
    return jnp.reshape(row, (row.shape[1], 1))


def _merge_into(L_ref, chunk):
    allv = jnp.concatenate([L_ref[...], chunk], axis=1)
    sc_row = allv[0:1, :]
    idx_row = allv[1:2, :]
    sc_col = _tocol(sc_row)
    idx_col = _tocol(idx_row)
    rank_col = jnp.zeros((2 * LCAP, 1), jnp.float32)
    for s in range(4):
        scs = sc_row[:, s * 512:(s + 1) * 512]
        idxs = idx_row[:, s * 512:(s + 1) * 512]
        before = ((scs > sc_col) |
                  ((scs == sc_col) & (idxs < idx_col))).astype(jnp.float32)
        rank_col = rank_col + jnp.sum(before, axis=1, keepdims=True)
    lane = jax.lax.broadcasted_iota(jnp.int32, (1, LCAP), 1).astype(jnp.float32)
    w = (rank_col == lane).astype(jnp.float32)
    L_ref[...] = jax.lax.dot_general(allv, w, (((1,), (0,)), ((), ())),
                                     preferred_element_type=jnp.float32)


def _nms_body(cnt_ref, sc_ref, idx_ref, x1_ref, y1_ref, x2_ref, y2_ref,
              dets_ref, num_ref, L_ref, iou_ref):
    wgrid = pl.program_id(0)
    lane = jax.lax.broadcasted_iota(jnp.int32, (1, LCAP), 1)
    lane_f = lane.astype(jnp.float32)

    @pl.when(wgrid == 0)
    def _init():
        L_ref[...] = jnp.concatenate(
            [jnp.zeros((1, LCAP), jnp.float32),
             EMPTY_IDX + lane_f,
             jnp.zeros((6, LCAP), jnp.float32)], axis=0)

    for k in range(RPG):
        cntw = cnt_ref[wgrid * RPG + k, 0]
        for c in range(CCAP // LCAP):
            @pl.when(cntw > c * LCAP)
            def _do_merge(c=c, k=k, cntw=cntw):
                rem = cntw - c * LCAP
                lm = lane < rem
                raw_sc = sc_ref[k, :, pl.ds(c * LCAP, LCAP)]
                raw_idx = idx_ref[k, :, pl.ds(c * LCAP, LCAP)].astype(
                    jnp.float32)
                csc = jnp.where(lm, raw_sc, -1.0)
                cidx = jnp.where(lm, raw_idx, INVAL_IDX + c * LCAP + lane_f)
                cx1 = jnp.where(lm, x1_ref[k, :, pl.ds(c * LCAP, LCAP)], 0.0)
                cy1 = jnp.where(lm, y1_ref[k, :, pl.ds(c * LCAP, LCAP)], 0.0)
                cx2 = jnp.where(lm, x2_ref[k, :, pl.ds(c * LCAP, LCAP)], 0.0)
                cy2 = jnp.where(lm, y2_ref[k, :, pl.ds(c * LCAP, LCAP)], 0.0)
                chunk = jnp.concatenate(
                    [csc, cidx, cx1, cy1, cx2, cy2,
                     jnp.zeros((2, LCAP), jnp.float32)], axis=0)
                _merge_into(L_ref, chunk)

    @pl.when(wgrid == NT // RPG - 1)
    def _final():
        L = L_ref[...]
        lsc = jnp.where(lane < PRE_NMS, L[0:1, :], 0.0)
        lidx = L[1:2, :]
        x1r = L[2:3, :]
        y1r = L[3:4, :]
        x2r = L[4:5, :]
        y2r = L[5:6, :]
        cls = lidx - jnp.floor(lidx * (1.0 / NUM_CLASSES)) * NUM_CLASSES
        off = cls * 4096.0
        ox1 = x1r + off
        oy1 = y1r + off
        ox2 = x2r + off
        oy2 = y2r + off
        ox1c = _tocol(ox1)
        oy1c = _tocol(oy1)
        ox2c = _tocol(ox2)
        oy2c = _tocol(oy2)
        area_r = jnp.maximum(ox2 - ox1, 0.0) * jnp.maximum(oy2 - oy1, 0.0)
        area_c = jnp.maximum(ox2c - ox1c, 0.0) * jnp.maximum(oy2c - oy1c, 0.0)
        ix1 = jnp.maximum(ox1c, ox1)
        iy1 = jnp.maximum(oy1c, oy1)
        ix2 = jnp.minimum(ox2c, ox2)
        iy2 = jnp.minimum(oy2c, oy2)
        inter = jnp.maximum(ix2 - ix1, 0.0) * jnp.maximum(iy2 - iy1, 0.0)
        iou_ref[...] = inter / (area_c + area_r - inter + 1e-6)

        npos = jnp.sum(jnp.where(lsc > 0.0, 1, 0))

        def body(i, keep):
            row = iou_ref[pl.ds(i, 1), :]
            ki = jnp.sum(jnp.where(lane == i, keep, 0.0))
            sup = (row > IOU_THR) & (lane > i) & (ki > 0.0)
            return jnp.where(sup, 0.0, keep)

        keep0 = jnp.where(lsc > 0.0, 1.0, 0.0)
        kept = jax.lax.fori_loop(0, npos, body, keep0)

        sub2d = jax.lax.broadcasted_iota(jnp.int32, (LCAP, LCAP), 0)
        lane2d = jax.lax.broadcasted_iota(jnp.int32, (LCAP, LCAP), 1)
        m3 = jnp.where(lane2d < sub2d, kept, 0.0)
        pr_col = jnp.sum(m3, axis=1, keepdims=True)
        lane128 = jax.lax.broadcasted_iota(
            jnp.int32, (1, 128), 1).astype(jnp.float32)
        wd = (pr_col == lane128).astype(jnp.float32)
        dmat = jnp.concatenate(
            [lsc, cls, x1r, y1r, x2r, y2r,
             jnp.zeros((2, LCAP), jnp.float32)], axis=0) * kept
        dets_ref[...] = jax.lax.dot_general(
            dmat, wd, (((1,), (0,)), ((), ())),
            preferred_element_type=jnp.float32)
        nk = jnp.sum(kept).astype(jnp.int32)
        num_ref[0, 0] = jnp.minimum(nk, MAX_PER_IMG)


def _run_nms(cnt2d, csc, cidx, cx1, cy1, cx2, cy2):
    cand_spec = pl.BlockSpec((RPG, 1, CCAP), lambda i: (i, 0, 0))
    return pl.pallas_call(
        _nms_body,
        grid=(NT // RPG,),
        in_specs=[
            pl.BlockSpec(memory_space=pltpu.SMEM),
            cand_spec, cand_spec, cand_spec, cand_spec, cand_spec, cand_spec,
        ],
        out_specs=[
            pl.BlockSpec((8, 128), lambda i: (0, 0)),
            pl.BlockSpec(memory_space=pltpu.SMEM),
        ],
        out_shape=[
            jax.ShapeDtypeStruct((8, 128), jnp.float32),
            jax.ShapeDtypeStruct((1, 1), jnp.int32),
        ],
        scratch_shapes=[
            pltpu.VMEM((8, LCAP), jnp.float32),
            pltpu.VMEM((LCAP, LCAP), jnp.float32),
        ],
        compiler_params=pltpu.CompilerParams(
            dimension_semantics=("arbitrary",)),
        interpret=_INTERP,
    )(cnt2d, csc, cidx, cx1, cy1, cx2, cy2)


def _sc_compact_body(scores_hbm, bx1_hbm, by1_hbm, bx2_hbm, by2_hbm,
                     cnt_out, sc_out, idx_out, x1_out, y1_out, x2_out, y2_out,
                     sc_v, bx1_v, by1_v, bx2_v, by2_v,
                     csc_v, cidx_v, cx1_v, cy1_v, cx2_v, cy2_v,
                     cnt_v):
    nc = 2
    wid = lax.axis_index("s") * nc + lax.axis_index("c")
    base = wid * TPT
    pltpu.sync_copy(scores_hbm.at[pl.ds(base, TPT)], sc_v)
    pltpu.sync_copy(bx1_hbm.at[pl.ds(base, TPT)], bx1_v)
    pltpu.sync_copy(by1_hbm.at[pl.ds(base, TPT)], by1_v)
    pltpu.sync_copy(bx2_hbm.at[pl.ds(base, TPT)], bx2_v)
    pltpu.sync_copy(by2_hbm.at[pl.ds(base, TPT)], by2_v)
    lane16 = lax.broadcasted_iota(jnp.int32, (16,), 0)

    def _prefix_inclusive(x):
        for d in (1, 2, 4, 8):
            src = jnp.maximum(lane16 - d, 0)
            y = x.at[src].get(mode="promise_in_bounds")
            x = x + jnp.where(lane16 >= d, y, 0.0)
        return x

    def row_body(r, cnt):
        for v in range(NUM_CLASSES // 16):
            s = sc_v[r, pl.ds(v * 16, 16)]
            m = s > SCORE_THR
            mf = jnp.where(m, 1.0, 0.0)
            csum = _prefix_inclusive(mf).astype(jnp.int32)
            pos = (cnt + csum) - 1
            gidx = (base + r) * NUM_CLASSES + v * 16 + lane16
            plsc.store_scatter(csc_v, [pos], s, mask=m)
            plsc.store_scatter(cidx_v, [pos], gidx, mask=m)
            x1v = bx1_v[r, pl.ds(v * 16, 16)]
            y1v = by1_v[r, pl.ds(v * 16, 16)]
            x2v = bx2_v[r, pl.ds(v * 16, 16)]
            y2v = by2_v[r, pl.ds(v * 16, 16)]
            plsc.store_scatter(cx1_v, [pos], x1v, mask=m)
            plsc.store_scatter(cy1_v, [pos], y1v, mask=m)
            plsc.store_scatter(cx2_v, [pos], x2v, mask=m)
            plsc.store_scatter(cy2_v, [pos], y2v, mask=m)
            last = csum.at[jnp.full((16,), 15, jnp.int32)].get(
                mode="promise_in_bounds")
            cnt = cnt + last
        return cnt

    cnt0 = jnp.zeros((16,), jnp.int32)
    cnt = lax.fori_loop(0, TPT, row_body, cnt0)
    cnt_v[...] = cnt
    pltpu.sync_copy(cnt_v, cnt_out.at[pl.ds(wid * 16, 16)])
    pltpu.sync_copy(csc_v, sc_out.at[wid])
    pltpu.sync_copy(cidx_v, idx_out.at[wid])
    pltpu.sync_copy(cx1_v, x1_out.at[wid])
    pltpu.sync_copy(cy1_v, y1_out.at[wid])
    pltpu.sync_copy(cx2_v, x2_out.at[wid])
    pltpu.sync_copy(cy2_v, y2_out.at[wid])


def _run_sc_compact(scores_pad, bx1, by1, bx2, by2):
    f32 = jnp.float32
    i32 = jnp.int32
    mesh = plsc.VectorSubcoreMesh(core_axis_name="c", subcore_axis_name="s")
    k = functools.partial(
        pl.kernel,
        mesh=mesh,
        out_type=[
            jax.ShapeDtypeStruct((NT * 16,), i32),
            jax.ShapeDtypeStruct((NT, CCAP), f32),
            jax.ShapeDtypeStruct((NT, CCAP), i32),
            jax.ShapeDtypeStruct((NT, CCAP), f32),
            jax.ShapeDtypeStruct((NT, CCAP), f32),
            jax.ShapeDtypeStruct((NT, CCAP), f32),
            jax.ShapeDtypeStruct((NT, CCAP), f32),
        ],
        scratch_types=[
            pltpu.VMEM((TPT, 128), f32),
            pltpu.VMEM((TPT, NUM_CLASSES), f32),
            pltpu.VMEM((TPT, NUM_CLASSES), f32),
            pltpu.VMEM((TPT, NUM_CLASSES), f32),
            pltpu.VMEM((TPT, NUM_CLASSES), f32),
            pltpu.VMEM((CCAP,), f32),
            pltpu.VMEM((CCAP,), i32),
            pltpu.VMEM((CCAP,), f32),
            pltpu.VMEM((CCAP,), f32),
            pltpu.VMEM((CCAP,), f32),
            pltpu.VMEM((CCAP,), f32),
            pltpu.VMEM((16,), i32),
        ],
        compiler_params=pltpu.CompilerParams(needs_layout_passes=False),
    )(_sc_compact_body)
    return k(scores_pad, bx1, by1, bx2, by2)


def _compact_emul_jax(scores_pad, bx1, by1, bx2, by2):
    sc3 = scores_pad[:, :NUM_CLASSES].reshape(NT, TPT * NUM_CLASSES)
    m = sc3 > SCORE_THR
    cnt = jnp.sum(m.astype(jnp.int32), axis=1)
    order = jnp.argsort(~m, axis=1, stable=True)[:, :CCAP]
    csc = jnp.take_along_axis(sc3, order, axis=1)
    base = (jnp.arange(NT, dtype=jnp.int32) * TPT * NUM_CLASSES)[:, None]
    cidx = base + order.astype(jnp.int32)
    outs = [jnp.take_along_axis(b.reshape(NT, TPT * NUM_CLASSES), order, axis=1)
            for b in (bx1, by1, bx2, by2)]
    return (cnt, csc, cidx, *outs)


def kernel(feat, proposals, W_cls, b_cls, W_reg, b_reg):
    ftx = jnp.transpose(feat[0], (2, 1, 0)).reshape(W, H * C)
    props_pad = jnp.pad(proposals[0], ((0, NPAD - N), (0, 0)))
    wcls_pad = jnp.pad(W_cls, ((0, 0), (0, 128 - (NUM_CLASSES + 1))))
    bcls_pad = jnp.pad(b_cls, (0, 128 - (NUM_CLASSES + 1))).reshape(1, 128)
    breg4 = jnp.transpose(b_reg.reshape(NUM_CLASSES, 4))
    wdx = W_reg[:, 0::4]
    wdy = W_reg[:, 1::4]
    wdw = W_reg[:, 2::4]
    wdh = W_reg[:, 3::4]

    scores_pad, bx1, by1, bx2, by2 = _run_head(
        props_pad, ftx, wcls_pad, bcls_pad, wdx, wdy, wdw, wdh, breg4)
    if _USE_SC:
        cntv, csc, cidx, cx1, cy1, cx2, cy2 = _run_sc_compact(
            scores_pad, bx1, by1, bx2, by2)
        cnt2d = cntv.reshape(NT, 16)
    else:
        cnt, csc, cidx, cx1, cy1, cx2, cy2 = _compact_emul_jax(
            scores_pad, bx1, by1, bx2, by2)
        cnt2d = jnp.broadcast_to(cnt[:, None], (NT, 16))
    shp3 = (NT, 1, CCAP)
    dets, num = _run_nms(cnt2d, csc.reshape(shp3), cidx.reshape(shp3),
                         cx1.reshape(shp3), cy1.reshape(shp3),
                         cx2.reshape(shp3), cy2.reshape(shp3))
    det_sc = dets[0, :MAX_PER_IMG]
    det_cls = jnp.where(det_sc > 0.0,
                        dets[1, :MAX_PER_IMG].astype(jnp.int32), -1)
    det_bx = jnp.transpose(dets[2:6, :MAX_PER_IMG])
    num_s = num[0, 0]
    return (num_s[None], det_bx[None], det_sc[None], det_cls[None])


# speedup vs baseline: 3.2429x; 1.1279x over previous
import functools

import jax
import jax.numpy as jnp
import numpy as np
from jax import lax
from jax.experimental import pallas as pl
from jax.experimental.pallas import tpu as pltpu
from jax.experimental.pallas import tpu_sc as plsc

NUM_CLASSES = 80
ROI = 7
STRIDE = 8
SCORE_THR = 0.05
IOU_THR = 0.5
MAX_PER_IMG = 100
PRE_NMS = 1000
H = 80
W = 80
C = 128
N = 5000
RB = 128
NPAD = 5120
NBLK = NPAD // RB
MAX_RATIO = float(np.abs(np.log(1000.0 / 16.0)))

_INTERP = False
_USE_SC = True


def _head_body(props_ref, ftx_ref, yexp_ref, wcls_ref, bcls_ref, wdx_ref,
               wdy_ref, wdw_ref, wdh_ref, breg_ref, scores_ref,
               bx1_ref, by1_ref, bx2_ref, by2_ref):
    props = props_ref[...]
    x1p = props[:, 0:1]
    y1p = props[:, 1:2]
    x2p = props[:, 2:3]
    y2p = props[:, 3:4]
    scale = 1.0 / STRIDE
    x1 = x1p * scale
    y1 = y1p * scale
    x2 = x2p * scale
    y2 = y2p * scale
    bw = jnp.maximum(x2 - x1, 1e-3) * (1.0 / ROI)
    bh = jnp.maximum(y2 - y1, 1e-3) * (1.0 / ROI)

    def samp_weights(lo, bsz):
        grid = jax.lax.broadcasted_iota(jnp.int32, (RB, W), 1).astype(jnp.float32)
        acc = jnp.zeros((RB, W), jnp.float32)
        for j in range(ROI):
            s = lo + (j + 0.5) * bsz
            f = jnp.floor(s)
            frac = s - f
            i0 = jnp.clip(f, 0.0, W - 1.0)
            i1 = jnp.clip(f + 1.0, 0.0, W - 1.0)
            acc = acc + jnp.where(grid == i0, 1.0 - frac, 0.0) \
                      + jnp.where(grid == i1, frac, 0.0)
        return acc * (1.0 / ROI)

    wx = samp_weights(x1, bw)
    wy = samp_weights(y1, bh)

    t = jax.lax.dot_general(wx, ftx_ref[...], (((1,), (0,)), ((), ())),
                            preferred_element_type=jnp.float32)
    wy_exp = jax.lax.dot_general(wy, yexp_ref[...], (((1,), (0,)), ((), ())),
                                 preferred_element_type=jnp.float32)
    prod = t * wy_exp
    parts = [prod[:, y * C:(y + 1) * C] for y in range(H)]
    while len(parts) > 1:
        nxt = [a + b for a, b in zip(parts[0::2], parts[1::2])]
        if len(parts) % 2:
            nxt[-1] = nxt[-1] + parts[-1]
        parts = nxt
    pooled = parts[0]

    logits = jax.lax.dot_general(pooled, wcls_ref[...], (((1,), (0,)), ((), ())),
                                 preferred_element_type=jnp.float32)
    logits = logits + bcls_ref[...]
    lane = jax.lax.broadcasted_iota(jnp.int32, (RB, 128), 1)
    logits = jnp.where(lane < NUM_CLASSES + 1, logits, -1e30)
    m = jnp.max(logits, axis=1, keepdims=True)
    e = jnp.exp(logits - m)
    ssum = jnp.sum(e, axis=1, keepdims=True)
    scores = e / ssum
    scores = jnp.where(lane < NUM_CLASSES + 1, scores, 0.0)
    gid = pl.program_id(0)
    row = gid * RB + jax.lax.broadcasted_iota(jnp.int32, (RB, 1), 0)
    scores = jnp.where(row < N, scores, 0.0)
    scores_ref[...] = scores

    def reg_head(w_ref, b_ref, std):
        d = jax.lax.dot_general(pooled, w_ref[...], (((1,), (0,)), ((), ())),
                                preferred_element_type=jnp.float32)
        return (d + b_ref[...]) * std

    dx = reg_head(wdx_ref, breg_ref.at[0:1], 0.1)
    dy = reg_head(wdy_ref, breg_ref.at[1:2], 0.1)
    dw = reg_head(wdw_ref, breg_ref.at[2:3], 0.2)
    dh = reg_head(wdh_ref, breg_ref.at[3:4], 0.2)
    dw = jnp.clip(dw, -MAX_RATIO, MAX_RATIO)
    dh = jnp.clip(dh, -MAX_RATIO, MAX_RATIO)

    px = (x1p + x2p) * 0.5
    py = (y1p + y2p) * 0.5
    pw = x2p - x1p
    ph = y2p - y1p
    gx = px + pw * dx
    gy = py + ph * dy
    gw = pw * jnp.exp(dw)
    gh = ph * jnp.exp(dh)
    bx1_ref[...] = gx - gw * 0.5
    by1_ref[...] = gy - gh * 0.5
    bx2_ref[...] = gx + gw * 0.5
    by2_ref[...] = gy + gh * 0.5


_YEXP = np.repeat(np.eye(H, dtype=np.float32), C, axis=1)


def _run_head(props_pad, ftx, wcls_pad, bcls_pad, wdx, wdy, wdw, wdh, breg4):
    full = lambda shape: pl.BlockSpec(shape, lambda i: tuple(0 for _ in shape))
    planar_out = pl.BlockSpec((RB, NUM_CLASSES), lambda i: (i, 0))
    return pl.pallas_call(
        _head_body,
        grid=(NBLK,),
        in_specs=[
            pl.BlockSpec((RB, 4), lambda i: (i, 0)),
            full((W, H * C)),
            full((H, H * C)),
            full((C, 128)),
            full((1, 128)),
            full((C, NUM_CLASSES)),
            full((C, NUM_CLASSES)),
            full((C, NUM_CLASSES)),
            full((C, NUM_CLASSES)),
            full((4, NUM_CLASSES)),
        ],
        out_specs=[
            pl.BlockSpec((RB, 128), lambda i: (i, 0)),
            planar_out, planar_out, planar_out, planar_out,
        ],
        out_shape=[
            jax.ShapeDtypeStruct((NPAD, 128), jnp.float32),
            jax.ShapeDtypeStruct((NPAD, NUM_CLASSES), jnp.float32),
            jax.ShapeDtypeStruct((NPAD, NUM_CLASSES), jnp.float32),
            jax.ShapeDtypeStruct((NPAD, NUM_CLASSES), jnp.float32),
            jax.ShapeDtypeStruct((NPAD, NUM_CLASSES), jnp.float32),
        ],
        compiler_params=pltpu.CompilerParams(
            dimension_semantics=("arbitrary",)),
        interpret=_INTERP,
    )(props_pad, ftx, jnp.asarray(_YEXP), wcls_pad, bcls_pad,
      wdx, wdy, wdw, wdh, breg4)


NT = 32
TPT = NPAD // NT
CCAP = 3072
LCAP = 1024
RPG = 1
EMPTY_IDX = 500000.0
INVAL_IDX = 600000.0


def _tocol(row):
    return jnp.reshape(row, (row.shape[1], 1))


def _merge_into(L_ref, chunk):
    allv = jnp.concatenate([L_ref[...], chunk], axis=1)
    sc_row = allv[0:1, :]
    idx_row = allv[1:2, :]
    sc_col = _tocol(sc_row)
    idx_col = _tocol(idx_row)
    rank_col = jnp.zeros((2 * LCAP, 1), jnp.float32)
    for s in range(4):
        scs = sc_row[:, s * 512:(s + 1) * 512]
        idxs = idx_row[:, s * 512:(s + 1) * 512]
        before = ((scs > sc_col) |
                  ((scs == sc_col) & (idxs < idx_col))).astype(jnp.float32)
        rank_col = rank_col + jnp.sum(before, axis=1, keepdims=True)
    lane = jax.lax.broadcasted_iota(jnp.int32, (1, LCAP), 1).astype(jnp.float32)
    w = (rank_col == lane).astype(jnp.float32)
    L_ref[...] = jax.lax.dot_general(allv, w, (((1,), (0,)), ((), ())),
                                     preferred_element_type=jnp.float32)


def _nms_body(cnt_ref, sc_ref, idx_ref, x1_ref, y1_ref, x2_ref, y2_ref,
              dets_ref, num_ref, L_ref, iou_ref,
              vsc_ref, vidx_ref, vx1_ref, vy1_ref, vx2_ref, vy2_ref, sem):
    lane = jax.lax.broadcasted_iota(jnp.int32, (1, LCAP), 1)
    lane_f = lane.astype(jnp.float32)

    L_ref[...] = jnp.concatenate(
        [jnp.zeros((1, LCAP), jnp.float32),
         EMPTY_IDX + lane_f,
         jnp.zeros((6, LCAP), jnp.float32)], axis=0)

    def region_body(wr, carry):
        cntw = cnt_ref[wr, 0]

        @pl.when(cntw > 0)
        def _do_region():
            pairs = [(sc_ref, vsc_ref), (idx_ref, vidx_ref),
                     (x1_ref, vx1_ref), (y1_ref, vy1_ref),
                     (x2_ref, vx2_ref), (y2_ref, vy2_ref)]
            copies = [pltpu.make_async_copy(src.at[pl.ds(wr, 1)], dst, sem)
                      for src, dst in pairs]
            for cp in copies:
                cp.start()
            for cp in copies:
                cp.wait()
            for c in range(CCAP // LCAP):
                @pl.when(cntw > c * LCAP)
                def _do_merge(c=c):
                    rem = cntw - c * LCAP
                    lm = lane < rem
                    raw_sc = vsc_ref[:, pl.ds(c * LCAP, LCAP)]
                    raw_idx = vidx_ref[:, pl.ds(c * LCAP, LCAP)].astype(
                        jnp.float32)
                    csc = jnp.where(lm, raw_sc, -1.0)
                    cidx = jnp.where(lm, raw_idx,
                                     INVAL_IDX + c * LCAP + lane_f)
                    cx1 = jnp.where(lm, vx1_ref[:, pl.ds(c * LCAP, LCAP)], 0.0)
                    cy1 = jnp.where(lm, vy1_ref[:, pl.ds(c * LCAP, LCAP)], 0.0)
                    cx2 = jnp.where(lm, vx2_ref[:, pl.ds(c * LCAP, LCAP)], 0.0)
                    cy2 = jnp.where(lm, vy2_ref[:, pl.ds(c * LCAP, LCAP)], 0.0)
                    chunk = jnp.concatenate(
                        [csc, cidx, cx1, cy1, cx2, cy2,
                         jnp.zeros((2, LCAP), jnp.float32)], axis=0)
                    _merge_into(L_ref, chunk)

        return carry

    jax.lax.fori_loop(0, NT, region_body, jnp.int32(0))

    def _final():
        L = L_ref[...]
        lsc = jnp.where(lane < PRE_NMS, L[0:1, :], 0.0)
        lidx = L[1:2, :]
        x1r = L[2:3, :]
        y1r = L[3:4, :]
        x2r = L[4:5, :]
        y2r = L[5:6, :]
        cls = lidx - jnp.floor(lidx * (1.0 / NUM_CLASSES)) * NUM_CLASSES
        off = cls * 4096.0
        ox1 = x1r + off
        oy1 = y1r + off
        ox2 = x2r + off
        oy2 = y2r + off
        ox1c = _tocol(ox1)
        oy1c = _tocol(oy1)
        ox2c = _tocol(ox2)
        oy2c = _tocol(oy2)
        area_r = jnp.maximum(ox2 - ox1, 0.0) * jnp.maximum(oy2 - oy1, 0.0)
        area_c = jnp.maximum(ox2c - ox1c, 0.0) * jnp.maximum(oy2c - oy1c, 0.0)
        ix1 = jnp.maximum(ox1c, ox1)
        iy1 = jnp.maximum(oy1c, oy1)
        ix2 = jnp.minimum(ox2c, ox2)
        iy2 = jnp.minimum(oy2c, oy2)
        inter = jnp.maximum(ix2 - ix1, 0.0) * jnp.maximum(iy2 - iy1, 0.0)
        iou_ref[...] = inter / (area_c + area_r - inter + 1e-6)

        npos = jnp.sum(jnp.where(lsc > 0.0, 1, 0))

        def body(i, keep):
            row = iou_ref[pl.ds(i, 1), :]
            ki = jnp.sum(jnp.where(lane == i, keep, 0.0))
            sup = (row > IOU_THR) & (lane > i) & (ki > 0.0)
            return jnp.where(sup, 0.0, keep)

        keep0 = jnp.where(lsc > 0.0, 1.0, 0.0)
        kept = jax.lax.fori_loop(0, npos, body, keep0)

        sub2d = jax.lax.broadcasted_iota(jnp.int32, (LCAP, LCAP), 0)
        lane2d = jax.lax.broadcasted_iota(jnp.int32, (LCAP, LCAP), 1)
        m3 = jnp.where(lane2d < sub2d, kept, 0.0)
        pr_col = jnp.sum(m3, axis=1, keepdims=True)
        lane128 = jax.lax.broadcasted_iota(
            jnp.int32, (1, 128), 1).astype(jnp.float32)
        wd = (pr_col == lane128).astype(jnp.float32)
        dmat = jnp.concatenate(
            [lsc, cls, x1r, y1r, x2r, y2r,
             jnp.zeros((2, LCAP), jnp.float32)], axis=0) * kept
        dets_ref[...] = jax.lax.dot_general(
            dmat, wd, (((1,), (0,)), ((), ())),
            preferred_element_type=jnp.float32)
        nk = jnp.sum(kept).astype(jnp.int32)
        num_ref[0, 0] = jnp.minimum(nk, MAX_PER_IMG)

    _final()


def _run_nms(cnt2d, csc, cidx, cx1, cy1, cx2, cy2):
    hbm_spec = pl.BlockSpec(memory_space=pl.ANY)
    return pl.pallas_call(
        _nms_body,
        in_specs=[
            pl.BlockSpec(memory_space=pltpu.SMEM),
            hbm_spec, hbm_spec, hbm_spec, hbm_spec, hbm_spec, hbm_spec,
        ],
        out_specs=[
            pl.BlockSpec((8, 128), lambda: (0, 0)),
            pl.BlockSpec(memory_space=pltpu.SMEM),
        ],
        out_shape=[
            jax.ShapeDtypeStruct((8, 128), jnp.float32),
            jax.ShapeDtypeStruct((1, 1), jnp.int32),
        ],
        scratch_shapes=[
            pltpu.VMEM((8, LCAP), jnp.float32),
            pltpu.VMEM((LCAP, LCAP), jnp.float32),
            pltpu.VMEM((1, CCAP), jnp.float32),
            pltpu.VMEM((1, CCAP), jnp.int32),
            pltpu.VMEM((1, CCAP), jnp.float32),
            pltpu.VMEM((1, CCAP), jnp.float32),
            pltpu.VMEM((1, CCAP), jnp.float32),
            pltpu.VMEM((1, CCAP), jnp.float32),
            pltpu.SemaphoreType.DMA,
        ],
        interpret=_INTERP,
    )(cnt2d, csc, cidx, cx1, cy1, cx2, cy2)


def _sc_compact_body(scores_hbm, bx1_hbm, by1_hbm, bx2_hbm, by2_hbm,
                     cnt_out, sc_out, idx_out, x1_out, y1_out, x2_out, y2_out,
                     sc_v, bx1_v, by1_v, bx2_v, by2_v,
                     csc_v, cidx_v, cx1_v, cy1_v, cx2_v, cy2_v,
                     cnt_v):
    nc = 2
    wid = lax.axis_index("s") * nc + lax.axis_index("c")
    base = wid * TPT
    pltpu.sync_copy(scores_hbm.at[pl.ds(base, TPT)], sc_v)
    pltpu.sync_copy(bx1_hbm.at[pl.ds(base, TPT)], bx1_v)
    pltpu.sync_copy(by1_hbm.at[pl.ds(base, TPT)], by1_v)
    pltpu.sync_copy(bx2_hbm.at[pl.ds(base, TPT)], bx2_v)
    pltpu.sync_copy(by2_hbm.at[pl.ds(base, TPT)], by2_v)
    lane16 = lax.broadcasted_iota(jnp.int32, (16,), 0)

    def _prefix_inclusive(x):
        for d in (1, 2, 4, 8):
            src = jnp.maximum(lane16 - d, 0)
            y = x.at[src].get(mode="promise_in_bounds")
            x = x + jnp.where(lane16 >= d, y, 0.0)
        return x

    def row_body(r, cnt):
        for v in range(NUM_CLASSES // 16):
            s = sc_v[r, pl.ds(v * 16, 16)]
            m = s > SCORE_THR
            mf = jnp.where(m, 1.0, 0.0)
            csum = _prefix_inclusive(mf).astype(jnp.int32)
            pos = (cnt + csum) - 1
            gidx = (base + r) * NUM_CLASSES + v * 16 + lane16
            plsc.store_scatter(csc_v, [pos], s, mask=m)
            plsc.store_scatter(cidx_v, [pos], gidx, mask=m)
            x1v = bx1_v[r, pl.ds(v * 16, 16)]
            y1v = by1_v[r, pl.ds(v * 16, 16)]
            x2v = bx2_v[r, pl.ds(v * 16, 16)]
            y2v = by2_v[r, pl.ds(v * 16, 16)]
            plsc.store_scatter(cx1_v, [pos], x1v, mask=m)
            plsc.store_scatter(cy1_v, [pos], y1v, mask=m)
            plsc.store_scatter(cx2_v, [pos], x2v, mask=m)
            plsc.store_scatter(cy2_v, [pos], y2v, mask=m)
            last = csum.at[jnp.full((16,), 15, jnp.int32)].get(
                mode="promise_in_bounds")
            cnt = cnt + last
        return cnt

    cnt0 = jnp.zeros((16,), jnp.int32)
    cnt = lax.fori_loop(0, TPT, row_body, cnt0)
    cnt_v[...] = cnt
    pltpu.sync_copy(cnt_v, cnt_out.at[pl.ds(wid * 16, 16)])
    pltpu.sync_copy(csc_v, sc_out.at[wid])
    pltpu.sync_copy(cidx_v, idx_out.at[wid])
    pltpu.sync_copy(cx1_v, x1_out.at[wid])
    pltpu.sync_copy(cy1_v, y1_out.at[wid])
    pltpu.sync_copy(cx2_v, x2_out.at[wid])
    pltpu.sync_copy(cy2_v, y2_out.at[wid])


def _run_sc_compact(scores_pad, bx1, by1, bx2, by2):
    f32 = jnp.float32
    i32 = jnp.int32
    mesh = plsc.VectorSubcoreMesh(core_axis_name="c", subcore_axis_name="s")
    k = functools.partial(
        pl.kernel,
        mesh=mesh,
        out_type=[
            jax.ShapeDtypeStruct((NT * 16,), i32),
            jax.ShapeDtypeStruct((NT, CCAP), f32),
            jax.ShapeDtypeStruct((NT, CCAP), i32),
            jax.ShapeDtypeStruct((NT, CCAP), f32),
            jax.ShapeDtypeStruct((NT, CCAP), f32),
            jax.ShapeDtypeStruct((NT, CCAP), f32),
            jax.ShapeDtypeStruct((NT, CCAP), f32),
        ],
        scratch_types=[
            pltpu.VMEM((TPT, 128), f32),
            pltpu.VMEM((TPT, NUM_CLASSES), f32),
            pltpu.VMEM((TPT, NUM_CLASSES), f32),
            pltpu.VMEM((TPT, NUM_CLASSES), f32),
            pltpu.VMEM((TPT, NUM_CLASSES), f32),
            pltpu.VMEM((CCAP,), f32),
            pltpu.VMEM((CCAP,), i32),
            pltpu.VMEM((CCAP,), f32),
            pltpu.VMEM((CCAP,), f32),
            pltpu.VMEM((CCAP,), f32),
            pltpu.VMEM((CCAP,), f32),
            pltpu.VMEM((16,), i32),
        ],
        compiler_params=pltpu.CompilerParams(needs_layout_passes=False),
    )(_sc_compact_body)
    return k(scores_pad, bx1, by1, bx2, by2)


def _compact_emul_jax(scores_pad, bx1, by1, bx2, by2):
    sc3 = scores_pad[:, :NUM_CLASSES].reshape(NT, TPT * NUM_CLASSES)
    m = sc3 > SCORE_THR
    cnt = jnp.sum(m.astype(jnp.int32), axis=1)
    order = jnp.argsort(~m, axis=1, stable=True)[:, :CCAP]
    csc = jnp.take_along_axis(sc3, order, axis=1)
    base = (jnp.arange(NT, dtype=jnp.int32) * TPT * NUM_CLASSES)[:, None]
    cidx = base + order.astype(jnp.int32)
    outs = [jnp.take_along_axis(b.reshape(NT, TPT * NUM_CLASSES), order, axis=1)
            for b in (bx1, by1, bx2, by2)]
    return (cnt, csc, cidx, *outs)


def kernel(feat, proposals, W_cls, b_cls, W_reg, b_reg):
    ftx = jnp.transpose(feat[0], (2, 1, 0)).reshape(W, H * C)
    props_pad = jnp.pad(proposals[0], ((0, NPAD - N), (0, 0)))
    wcls_pad = jnp.pad(W_cls, ((0, 0), (0, 128 - (NUM_CLASSES + 1))))
    bcls_pad = jnp.pad(b_cls, (0, 128 - (NUM_CLASSES + 1))).reshape(1, 128)
    breg4 = jnp.transpose(b_reg.reshape(NUM_CLASSES, 4))
    wdx = W_reg[:, 0::4]
    wdy = W_reg[:, 1::4]
    wdw = W_reg[:, 2::4]
    wdh = W_reg[:, 3::4]

    scores_pad, bx1, by1, bx2, by2 = _run_head(
        props_pad, ftx, wcls_pad, bcls_pad, wdx, wdy, wdw, wdh, breg4)
    if _USE_SC:
        cntv, csc, cidx, cx1, cy1, cx2, cy2 = _run_sc_compact(
            scores_pad, bx1, by1, bx2, by2)
        cnt2d = cntv.reshape(NT, 16)
    else:
        cnt, csc, cidx, cx1, cy1, cx2, cy2 = _compact_emul_jax(
            scores_pad, bx1, by1, bx2, by2)
        cnt2d = jnp.broadcast_to(cnt[:, None], (NT, 16))
    dets, num = _run_nms(cnt2d, csc, cidx, cx1, cy1, cx2, cy2)
    det_sc = dets[0, :MAX_PER_IMG]
    det_cls = jnp.where(det_sc > 0.0,
                        dets[1, :MAX_PER_IMG].astype(jnp.int32), -1)
    det_bx = jnp.transpose(dets[2:6, :MAX_PER_IMG])
    num_s = num[0, 0]
    return (num_s[None], det_bx[None], det_sc[None], det_cls[None])


# speedup vs baseline: 3.5820x; 1.1046x over previous
import functools

import jax
import jax.numpy as jnp
import numpy as np
from jax import lax
from jax.experimental import pallas as pl
from jax.experimental.pallas import tpu as pltpu
from jax.experimental.pallas import tpu_sc as plsc

NUM_CLASSES = 80
ROI = 7
STRIDE = 8
SCORE_THR = 0.05
IOU_THR = 0.5
MAX_PER_IMG = 100
PRE_NMS = 1000
H = 80
W = 80
C = 128
N = 5000
RB = 128
NPAD = 5120
NBLK = NPAD // RB
MAX_RATIO = float(np.abs(np.log(1000.0 / 16.0)))

_INTERP = False
_USE_SC = True


def _head_body(props_ref, ftx_ref, yexp_ref, wcls_ref, bcls_ref, wdx_ref,
               wdy_ref, wdw_ref, wdh_ref, breg_ref, scores_ref,
               bx1_ref, by1_ref, bx2_ref, by2_ref):
    props = props_ref[...]
    x1p = props[:, 0:1]
    y1p = props[:, 1:2]
    x2p = props[:, 2:3]
    y2p = props[:, 3:4]
    scale = 1.0 / STRIDE
    x1 = x1p * scale
    y1 = y1p * scale
    x2 = x2p * scale
    y2 = y2p * scale
    bw = jnp.maximum(x2 - x1, 1e-3) * (1.0 / ROI)
    bh = jnp.maximum(y2 - y1, 1e-3) * (1.0 / ROI)

    def samp_weights(lo, bsz):
        grid = jax.lax.broadcasted_iota(jnp.int32, (RB, W), 1).astype(jnp.float32)
        acc = jnp.zeros((RB, W), jnp.float32)
        for j in range(ROI):
            s = lo + (j + 0.5) * bsz
            f = jnp.floor(s)
            frac = s - f
            i0 = jnp.clip(f, 0.0, W - 1.0)
            i1 = jnp.clip(f + 1.0, 0.0, W - 1.0)
            acc = acc + jnp.where(grid == i0, 1.0 - frac, 0.0) \
                      + jnp.where(grid == i1, frac, 0.0)
        return acc * (1.0 / ROI)

    wx = samp_weights(x1, bw)
    wy = samp_weights(y1, bh)

    t = jax.lax.dot_general(wx, ftx_ref[...], (((1,), (0,)), ((), ())),
                            preferred_element_type=jnp.float32)
    wy_exp = jax.lax.dot_general(wy, yexp_ref[...], (((1,), (0,)), ((), ())),
                                 preferred_element_type=jnp.float32)
    prod = t * wy_exp
    parts = [prod[:, y * C:(y + 1) * C] for y in range(H)]
    while len(parts) > 1:
        nxt = [a + b for a, b in zip(parts[0::2], parts[1::2])]
        if len(parts) % 2:
            nxt[-1] = nxt[-1] + parts[-1]
        parts = nxt
    pooled = parts[0]

    logits = jax.lax.dot_general(pooled, wcls_ref[...], (((1,), (0,)), ((), ())),
                                 preferred_element_type=jnp.float32)
    logits = logits + bcls_ref[...]
    lane = jax.lax.broadcasted_iota(jnp.int32, (RB, 128), 1)
    logits = jnp.where(lane < NUM_CLASSES + 1, logits, -1e30)
    m = jnp.max(logits, axis=1, keepdims=True)
    e = jnp.exp(logits - m)
    ssum = jnp.sum(e, axis=1, keepdims=True)
    scores = e / ssum
    gid = pl.program_id(0)
    row = gid * RB + jax.lax.broadcasted_iota(jnp.int32, (RB, 1), 0)
    scores = jnp.where(row < N, scores, 0.0)
    scores_ref[...] = scores[:, :NUM_CLASSES]

    def reg_head(w_ref, b_ref, std):
        d = jax.lax.dot_general(pooled, w_ref[...], (((1,), (0,)), ((), ())),
                                preferred_element_type=jnp.float32)
        return (d + b_ref[...]) * std

    dx = reg_head(wdx_ref, breg_ref.at[0:1], 0.1)
    dy = reg_head(wdy_ref, breg_ref.at[1:2], 0.1)
    dw = reg_head(wdw_ref, breg_ref.at[2:3], 0.2)
    dh = reg_head(wdh_ref, breg_ref.at[3:4], 0.2)
    dw = jnp.clip(dw, -MAX_RATIO, MAX_RATIO)
    dh = jnp.clip(dh, -MAX_RATIO, MAX_RATIO)

    px = (x1p + x2p) * 0.5
    py = (y1p + y2p) * 0.5
    pw = x2p - x1p
    ph = y2p - y1p
    gx = px + pw * dx
    gy = py + ph * dy
    gw = pw * jnp.exp(dw)
    gh = ph * jnp.exp(dh)
    bx1_ref[...] = gx - gw * 0.5
    by1_ref[...] = gy - gh * 0.5
    bx2_ref[...] = gx + gw * 0.5
    by2_ref[...] = gy + gh * 0.5


_YEXP = np.repeat(np.eye(H, dtype=np.float32), C, axis=1)


def _run_head(props_pad, ftx, wcls_pad, bcls_pad, wdx, wdy, wdw, wdh, breg4):
    full = lambda shape: pl.BlockSpec(shape, lambda i: tuple(0 for _ in shape))
    planar_out = pl.BlockSpec((RB, NUM_CLASSES), lambda i: (i, 0))
    return pl.pallas_call(
        _head_body,
        grid=(NBLK,),
        in_specs=[
            pl.BlockSpec((RB, 4), lambda i: (i, 0)),
            full((W, H * C)),
            full((H, H * C)),
            full((C, 128)),
            full((1, 128)),
            full((C, NUM_CLASSES)),
            full((C, NUM_CLASSES)),
            full((C, NUM_CLASSES)),
            full((C, NUM_CLASSES)),
            full((4, NUM_CLASSES)),
        ],
        out_specs=[
            planar_out,
            planar_out, planar_out, planar_out, planar_out,
        ],
        out_shape=[
            jax.ShapeDtypeStruct((NPAD, NUM_CLASSES), jnp.float32),
            jax.ShapeDtypeStruct((NPAD, NUM_CLASSES), jnp.float32),
            jax.ShapeDtypeStruct((NPAD, NUM_CLASSES), jnp.float32),
            jax.ShapeDtypeStruct((NPAD, NUM_CLASSES), jnp.float32),
            jax.ShapeDtypeStruct((NPAD, NUM_CLASSES), jnp.float32),
        ],
        compiler_params=pltpu.CompilerParams(
            dimension_semantics=("arbitrary",)),
        interpret=_INTERP,
    )(props_pad, ftx, jnp.asarray(_YEXP), wcls_pad, bcls_pad,
      wdx, wdy, wdw, wdh, breg4)


NT = 32
TPT = NPAD // NT
CCAP = 3072
LCAP = 1024
RPG = 1
EMPTY_IDX = 500000.0
INVAL_IDX = 600000.0


def _tocol(row):
    return jnp.reshape(row, (row.shape[1], 1))


def _merge_into(L_ref, chunk):
    allv = jnp.concatenate([L_ref[...], chunk], axis=1)
    sc_row = allv[0:1, :]
    idx_row = allv[1:2, :]
    sc_col = _tocol(sc_row)
    idx_col = _tocol(idx_row)
    rank_col = jnp.zeros((2 * LCAP, 1), jnp.float32)
    for s in range(4):
        scs = sc_row[:, s * 512:(s + 1) * 512]
        idxs = idx_row[:, s * 512:(s + 1) * 512]
        before = ((scs > sc_col) |
                  ((scs == sc_col) & (idxs < idx_col))).astype(jnp.float32)
        rank_col = rank_col + jnp.sum(before, axis=1, keepdims=True)
    lane = jax.lax.broadcasted_iota(jnp.int32, (1, LCAP), 1).astype(jnp.float32)
    w = (rank_col == lane).astype(jnp.float32)
    L_ref[...] = jax.lax.dot_general(allv, w, (((1,), (0,)), ((), ())),
                                     preferred_element_type=jnp.float32)


def _nms_body(cnt_ref, sc_ref, idx_ref, x1_ref, y1_ref, x2_ref, y2_ref,
              dets_ref, num_ref, L_ref, iou_ref,
              vsc_ref, vidx_ref, vx1_ref, vy1_ref, vx2_ref, vy2_ref, sem):
    lane = jax.lax.broadcasted_iota(jnp.int32, (1, LCAP), 1)
    lane_f = lane.astype(jnp.float32)

    L_ref[...] = jnp.concatenate(
        [jnp.zeros((1, LCAP), jnp.float32),
         EMPTY_IDX + lane_f,
         jnp.zeros((6, LCAP), jnp.float32)], axis=0)

    def region_body(wr, carry):
        cntw = cnt_ref[wr, 0]

        @pl.when(cntw > 0)
        def _do_region():
            pairs = [(sc_ref, vsc_ref), (idx_ref, vidx_ref),
                     (x1_ref, vx1_ref), (y1_ref, vy1_ref),
                     (x2_ref, vx2_ref), (y2_ref, vy2_ref)]
            copies = [pltpu.make_async_copy(src.at[pl.ds(wr, 1)], dst, sem)
                      for src, dst in pairs]
            for cp in copies:
                cp.start()
            for cp in copies:
                cp.wait()
            for c in range(CCAP // LCAP):
                @pl.when(cntw > c * LCAP)
                def _do_merge(c=c):
                    rem = cntw - c * LCAP
                    lm = lane < rem
                    raw_sc = vsc_ref[:, pl.ds(c * LCAP, LCAP)]
                    raw_idx = vidx_ref[:, pl.ds(c * LCAP, LCAP)].astype(
                        jnp.float32)
                    csc = jnp.where(lm, raw_sc, -1.0)
                    cidx = jnp.where(lm, raw_idx,
                                     INVAL_IDX + c * LCAP + lane_f)
                    cx1 = jnp.where(lm, vx1_ref[:, pl.ds(c * LCAP, LCAP)], 0.0)
                    cy1 = jnp.where(lm, vy1_ref[:, pl.ds(c * LCAP, LCAP)], 0.0)
                    cx2 = jnp.where(lm, vx2_ref[:, pl.ds(c * LCAP, LCAP)], 0.0)
                    cy2 = jnp.where(lm, vy2_ref[:, pl.ds(c * LCAP, LCAP)], 0.0)
                    chunk = jnp.concatenate(
                        [csc, cidx, cx1, cy1, cx2, cy2,
                         jnp.zeros((2, LCAP), jnp.float32)], axis=0)
                    _merge_into(L_ref, chunk)

        return carry

    jax.lax.fori_loop(0, NT, region_body, jnp.int32(0))

    def _final():
        L = L_ref[...]
        lsc = jnp.where(lane < PRE_NMS, L[0:1, :], 0.0)
        lidx = L[1:2, :]
        x1r = L[2:3, :]
        y1r = L[3:4, :]
        x2r = L[4:5, :]
        y2r = L[5:6, :]
        cls = lidx - jnp.floor(lidx * (1.0 / NUM_CLASSES)) * NUM_CLASSES
        off = cls * 4096.0
        ox1 = x1r + off
        oy1 = y1r + off
        ox2 = x2r + off
        oy2 = y2r + off
        ox1c = _tocol(ox1)
        oy1c = _tocol(oy1)
        ox2c = _tocol(ox2)
        oy2c = _tocol(oy2)
        area_r = jnp.maximum(ox2 - ox1, 0.0) * jnp.maximum(oy2 - oy1, 0.0)
        area_c = jnp.maximum(ox2c - ox1c, 0.0) * jnp.maximum(oy2c - oy1c, 0.0)
        ix1 = jnp.maximum(ox1c, ox1)
        iy1 = jnp.maximum(oy1c, oy1)
        ix2 = jnp.minimum(ox2c, ox2)
        iy2 = jnp.minimum(oy2c, oy2)
        inter = jnp.maximum(ix2 - ix1, 0.0) * jnp.maximum(iy2 - iy1, 0.0)
        iou_ref[...] = inter / (area_c + area_r - inter + 1e-6)

        npos = jnp.sum(jnp.where(lsc > 0.0, 1, 0))

        def body(i, keep):
            row = iou_ref[pl.ds(i, 1), :]
            ki = jnp.sum(jnp.where(lane == i, keep, 0.0))
            sup = (row > IOU_THR) & (lane > i) & (ki > 0.0)
            return jnp.where(sup, 0.0, keep)

        keep0 = jnp.where(lsc > 0.0, 1.0, 0.0)
        kept = jax.lax.fori_loop(0, npos, body, keep0)

        sub2d = jax.lax.broadcasted_iota(jnp.int32, (LCAP, LCAP), 0)
        lane2d = jax.lax.broadcasted_iota(jnp.int32, (LCAP, LCAP), 1)
        m3 = jnp.where(lane2d < sub2d, kept, 0.0)
        pr_col = jnp.sum(m3, axis=1, keepdims=True)
        lane128 = jax.lax.broadcasted_iota(
            jnp.int32, (1, 128), 1).astype(jnp.float32)
        wd = (pr_col == lane128).astype(jnp.float32)
        dmat = jnp.concatenate(
            [lsc, cls, x1r, y1r, x2r, y2r,
             jnp.zeros((2, LCAP), jnp.float32)], axis=0) * kept
        dets_ref[...] = jax.lax.dot_general(
            dmat, wd, (((1,), (0,)), ((), ())),
            preferred_element_type=jnp.float32)
        nk = jnp.sum(kept).astype(jnp.int32)
        num_ref[0, 0] = jnp.minimum(nk, MAX_PER_IMG)

    _final()


def _run_nms(cnt2d, csc, cidx, cx1, cy1, cx2, cy2):
    hbm_spec = pl.BlockSpec(memory_space=pl.ANY)
    return pl.pallas_call(
        _nms_body,
        in_specs=[
            pl.BlockSpec(memory_space=pltpu.SMEM),
            hbm_spec, hbm_spec, hbm_spec, hbm_spec, hbm_spec, hbm_spec,
        ],
        out_specs=[
            pl.BlockSpec((8, 128), lambda: (0, 0)),
            pl.BlockSpec(memory_space=pltpu.SMEM),
        ],
        out_shape=[
            jax.ShapeDtypeStruct((8, 128), jnp.float32),
            jax.ShapeDtypeStruct((1, 1), jnp.int32),
        ],
        scratch_shapes=[
            pltpu.VMEM((8, LCAP), jnp.float32),
            pltpu.VMEM((LCAP, LCAP), jnp.float32),
            pltpu.VMEM((1, CCAP), jnp.float32),
            pltpu.VMEM((1, CCAP), jnp.int32),
            pltpu.VMEM((1, CCAP), jnp.float32),
            pltpu.VMEM((1, CCAP), jnp.float32),
            pltpu.VMEM((1, CCAP), jnp.float32),
            pltpu.VMEM((1, CCAP), jnp.float32),
            pltpu.SemaphoreType.DMA,
        ],
        interpret=_INTERP,
    )(cnt2d, csc, cidx, cx1, cy1, cx2, cy2)


def _sc_compact_body(scores_hbm, bx1_hbm, by1_hbm, bx2_hbm, by2_hbm,
                     cnt_out, sc_out, idx_out, x1_out, y1_out, x2_out, y2_out,
                     sc_v, bx1_v, by1_v, bx2_v, by2_v,
                     csc_v, cidx_v, cx1_v, cy1_v, cx2_v, cy2_v,
                     cnt_v, cnt_s):
    nc = 2
    wid = lax.axis_index("s") * nc + lax.axis_index("c")
    base = wid * TPT
    pltpu.sync_copy(scores_hbm.at[pl.ds(base, TPT)], sc_v)
    pltpu.sync_copy(bx1_hbm.at[pl.ds(base, TPT)], bx1_v)
    pltpu.sync_copy(by1_hbm.at[pl.ds(base, TPT)], by1_v)
    pltpu.sync_copy(bx2_hbm.at[pl.ds(base, TPT)], bx2_v)
    pltpu.sync_copy(by2_hbm.at[pl.ds(base, TPT)], by2_v)
    lane16 = lax.broadcasted_iota(jnp.int32, (16,), 0)
    cnt_s[0] = 0

    def _prefix_inclusive(x):
        for d in (1, 2, 4, 8):
            src = jnp.maximum(lane16 - d, 0)
            y = x.at[src].get(mode="promise_in_bounds")
            x = x + jnp.where(lane16 >= d, y, 0.0)
        return x

    def row_body(r, carry):
        svals = [sc_v[r, pl.ds(v * 16, 16)] for v in range(NUM_CLASSES // 16)]
        masks = [s > SCORE_THR for s in svals]
        many = (masks[0] | masks[1]) | (masks[2] | masks[3]) | masks[4]
        npos_any = plsc.all_reduce_population_count(many)[0]

        @pl.when(npos_any > 0)
        def _process_row():
            cnt = jnp.full((16,), cnt_s[0], jnp.int32)
            for v in range(NUM_CLASSES // 16):
                s = svals[v]
                m = masks[v]
                mf = jnp.where(m, 1.0, 0.0)
                csum = _prefix_inclusive(mf).astype(jnp.int32)
                pos = (cnt + csum) - 1
                gidx = (base + r) * NUM_CLASSES + v * 16 + lane16
                plsc.store_scatter(csc_v, [pos], s, mask=m)
                plsc.store_scatter(cidx_v, [pos], gidx, mask=m)
                plsc.store_scatter(cx1_v, [pos], bx1_v[r, pl.ds(v * 16, 16)],
                                   mask=m)
                plsc.store_scatter(cy1_v, [pos], by1_v[r, pl.ds(v * 16, 16)],
                                   mask=m)
                plsc.store_scatter(cx2_v, [pos], bx2_v[r, pl.ds(v * 16, 16)],
                                   mask=m)
                plsc.store_scatter(cy2_v, [pos], by2_v[r, pl.ds(v * 16, 16)],
                                   mask=m)
                last = csum.at[jnp.full((16,), 15, jnp.int32)].get(
                    mode="promise_in_bounds")
                cnt = cnt + last
            cnt_s[0] = cnt[0]

        return carry

    lax.fori_loop(0, TPT, row_body, jnp.int32(0))
    cnt_v[...] = jnp.full((16,), cnt_s[0], jnp.int32)
    pltpu.sync_copy(cnt_v, cnt_out.at[pl.ds(wid * 16, 16)])
    pltpu.sync_copy(csc_v, sc_out.at[wid])
    pltpu.sync_copy(cidx_v, idx_out.at[wid])
    pltpu.sync_copy(cx1_v, x1_out.at[wid])
    pltpu.sync_copy(cy1_v, y1_out.at[wid])
    pltpu.sync_copy(cx2_v, x2_out.at[wid])
    pltpu.sync_copy(cy2_v, y2_out.at[wid])


def _run_sc_compact(scores_pad, bx1, by1, bx2, by2):
    f32 = jnp.float32
    i32 = jnp.int32
    mesh = plsc.VectorSubcoreMesh(core_axis_name="c", subcore_axis_name="s")
    k = functools.partial(
        pl.kernel,
        mesh=mesh,
        out_type=[
            jax.ShapeDtypeStruct((NT * 16,), i32),
            jax.ShapeDtypeStruct((NT, CCAP), f32),
            jax.ShapeDtypeStruct((NT, CCAP), i32),
            jax.ShapeDtypeStruct((NT, CCAP), f32),
            jax.ShapeDtypeStruct((NT, CCAP), f32),
            jax.ShapeDtypeStruct((NT, CCAP), f32),
            jax.ShapeDtypeStruct((NT, CCAP), f32),
        ],
        scratch_types=[
            pltpu.VMEM((TPT, NUM_CLASSES), f32),
            pltpu.VMEM((TPT, NUM_CLASSES), f32),
            pltpu.VMEM((TPT, NUM_CLASSES), f32),
            pltpu.VMEM((TPT, NUM_CLASSES), f32),
            pltpu.VMEM((TPT, NUM_CLASSES), f32),
            pltpu.VMEM((CCAP,), f32),
            pltpu.VMEM((CCAP,), i32),
            pltpu.VMEM((CCAP,), f32),
            pltpu.VMEM((CCAP,), f32),
            pltpu.VMEM((CCAP,), f32),
            pltpu.VMEM((CCAP,), f32),
            pltpu.VMEM((16,), i32),
            pltpu.SMEM((1,), i32),
        ],
        compiler_params=pltpu.CompilerParams(needs_layout_passes=False),
    )(_sc_compact_body)
    return k(scores_pad, bx1, by1, bx2, by2)


def _compact_emul_jax(scores_pad, bx1, by1, bx2, by2):
    sc3 = scores_pad.reshape(NT, TPT * NUM_CLASSES)
    m = sc3 > SCORE_THR
    cnt = jnp.sum(m.astype(jnp.int32), axis=1)
    order = jnp.argsort(~m, axis=1, stable=True)[:, :CCAP]
    csc = jnp.take_along_axis(sc3, order, axis=1)
    base = (jnp.arange(NT, dtype=jnp.int32) * TPT * NUM_CLASSES)[:, None]
    cidx = base + order.astype(jnp.int32)
    outs = [jnp.take_along_axis(b.reshape(NT, TPT * NUM_CLASSES), order, axis=1)
            for b in (bx1, by1, bx2, by2)]
    return (cnt, csc, cidx, *outs)


def kernel(feat, proposals, W_cls, b_cls, W_reg, b_reg):
    ftx = jnp.transpose(feat[0], (2, 1, 0)).reshape(W, H * C)
    props_pad = jnp.pad(proposals[0], ((0, NPAD - N), (0, 0)))
    wcls_pad = jnp.pad(W_cls, ((0, 0), (0, 128 - (NUM_CLASSES + 1))))
    bcls_pad = jnp.pad(b_cls, (0, 128 - (NUM_CLASSES + 1))).reshape(1, 128)
    breg4 = jnp.transpose(b_reg.reshape(NUM_CLASSES, 4))
    wdx = W_reg[:, 0::4]
    wdy = W_reg[:, 1::4]
    wdw = W_reg[:, 2::4]
    wdh = W_reg[:, 3::4]

    scores_pad, bx1, by1, bx2, by2 = _run_head(
        props_pad, ftx, wcls_pad, bcls_pad, wdx, wdy, wdw, wdh, breg4)
    if _USE_SC:
        cntv, csc, cidx, cx1, cy1, cx2, cy2 = _run_sc_compact(
            scores_pad, bx1, by1, bx2, by2)
        cnt2d = cntv.reshape(NT, 16)
    else:
        cnt, csc, cidx, cx1, cy1, cx2, cy2 = _compact_emul_jax(
            scores_pad, bx1, by1, bx2, by2)
        cnt2d = jnp.broadcast_to(cnt[:, None], (NT, 16))
    dets, num = _run_nms(cnt2d, csc, cidx, cx1, cy1, cx2, cy2)
    det_sc = dets[0, :MAX_PER_IMG]
    det_cls = jnp.where(det_sc > 0.0,
                        dets[1, :MAX_PER_IMG].astype(jnp.int32), -1)
    det_bx = jnp.transpose(dets[2:6, :MAX_PER_IMG])
    num_s = num[0, 0]
    return (num_s[None], det_bx[None], det_sc[None], det_cls[None])


# speedup vs baseline: 4.0558x; 1.1323x over previous
import functools

import jax
import jax.numpy as jnp
import numpy as np
from jax import lax
from jax.experimental import pallas as pl
from jax.experimental.pallas import tpu as pltpu
from jax.experimental.pallas import tpu_sc as plsc

NUM_CLASSES = 80
ROI = 7
STRIDE = 8
SCORE_THR = 0.05
IOU_THR = 0.5
MAX_PER_IMG = 100
PRE_NMS = 1000
H = 80
W = 80
C = 128
N = 5000
RB = 512
NPAD = 5120
NBLK = NPAD // RB
MAX_RATIO = float(np.abs(np.log(1000.0 / 16.0)))

_INTERP = False
_USE_SC = True


def _head_body(props_ref, ftx_ref, yexp_ref, wcls_ref, bcls_ref, wdx_ref,
               wdy_ref, wdw_ref, wdh_ref, breg_ref, scores_ref,
               bx1_ref, by1_ref, bx2_ref, by2_ref):
    props = props_ref[...]
    x1p = props[:, 0:1]
    y1p = props[:, 1:2]
    x2p = props[:, 2:3]
    y2p = props[:, 3:4]
    scale = 1.0 / STRIDE
    x1 = x1p * scale
    y1 = y1p * scale
    x2 = x2p * scale
    y2 = y2p * scale
    bw = jnp.maximum(x2 - x1, 1e-3) * (1.0 / ROI)
    bh = jnp.maximum(y2 - y1, 1e-3) * (1.0 / ROI)

    def samp_weights(lo, bsz):
        grid = jax.lax.broadcasted_iota(jnp.int32, (RB, W), 1).astype(jnp.float32)
        acc = jnp.zeros((RB, W), jnp.float32)
        for j in range(ROI):
            s = lo + (j + 0.5) * bsz
            f = jnp.floor(s)
            frac = s - f
            i0 = jnp.clip(f, 0.0, W - 1.0)
            i1 = jnp.clip(f + 1.0, 0.0, W - 1.0)
            acc = acc + jnp.where(grid == i0, 1.0 - frac, 0.0) \
                      + jnp.where(grid == i1, frac, 0.0)
        return acc * (1.0 / ROI)

    wx = samp_weights(x1, bw)
    wy = samp_weights(y1, bh)

    t = jax.lax.dot_general(wx, ftx_ref[...], (((1,), (0,)), ((), ())),
                            preferred_element_type=jnp.float32)
    wy_exp = jax.lax.dot_general(wy, yexp_ref[...], (((1,), (0,)), ((), ())),
                                 preferred_element_type=jnp.float32)
    prod = t * wy_exp
    parts = [prod[:, y * C:(y + 1) * C] for y in range(H)]
    while len(parts) > 1:
        nxt = [a + b for a, b in zip(parts[0::2], parts[1::2])]
        if len(parts) % 2:
            nxt[-1] = nxt[-1] + parts[-1]
        parts = nxt
    pooled = parts[0]

    logits = jax.lax.dot_general(pooled, wcls_ref[...], (((1,), (0,)), ((), ())),
                                 preferred_element_type=jnp.float32)
    logits = logits + bcls_ref[...]
    lane = jax.lax.broadcasted_iota(jnp.int32, (RB, 128), 1)
    logits = jnp.where(lane < NUM_CLASSES + 1, logits, -1e30)
    m = jnp.max(logits, axis=1, keepdims=True)
    e = jnp.exp(logits - m)
    ssum = jnp.sum(e, axis=1, keepdims=True)
    scores = e / ssum
    gid = pl.program_id(0)
    row = gid * RB + jax.lax.broadcasted_iota(jnp.int32, (RB, 1), 0)
    scores = jnp.where(row < N, scores, 0.0)
    scores_ref[...] = scores[:, :NUM_CLASSES]

    def reg_head(w_ref, b_ref, std):
        d = jax.lax.dot_general(pooled, w_ref[...], (((1,), (0,)), ((), ())),
                                preferred_element_type=jnp.float32)
        return (d + b_ref[...]) * std

    dx = reg_head(wdx_ref, breg_ref.at[0:1], 0.1)
    dy = reg_head(wdy_ref, breg_ref.at[1:2], 0.1)
    dw = reg_head(wdw_ref, breg_ref.at[2:3], 0.2)
    dh = reg_head(wdh_ref, breg_ref.at[3:4], 0.2)
    dw = jnp.clip(dw, -MAX_RATIO, MAX_RATIO)
    dh = jnp.clip(dh, -MAX_RATIO, MAX_RATIO)

    px = (x1p + x2p) * 0.5
    py = (y1p + y2p) * 0.5
    pw = x2p - x1p
    ph = y2p - y1p
    gx = px + pw * dx
    gy = py + ph * dy
    gw = pw * jnp.exp(dw)
    gh = ph * jnp.exp(dh)
    bx1_ref[...] = gx - gw * 0.5
    by1_ref[...] = gy - gh * 0.5
    bx2_ref[...] = gx + gw * 0.5
    by2_ref[...] = gy + gh * 0.5


_YEXP = np.repeat(np.eye(H, dtype=np.float32), C, axis=1)


def _run_head(props_pad, ftx, wcls_pad, bcls_pad, wdx, wdy, wdw, wdh, breg4):
    full = lambda shape: pl.BlockSpec(shape, lambda i: tuple(0 for _ in shape))
    planar_out = pl.BlockSpec((RB, NUM_CLASSES), lambda i: (i, 0))
    return pl.pallas_call(
        _head_body,
        grid=(NBLK,),
        in_specs=[
            pl.BlockSpec((RB, 4), lambda i: (i, 0)),
            full((W, H * C)),
            full((H, H * C)),
            full((C, 128)),
            full((1, 128)),
            full((C, NUM_CLASSES)),
            full((C, NUM_CLASSES)),
            full((C, NUM_CLASSES)),
            full((C, NUM_CLASSES)),
            full((4, NUM_CLASSES)),
        ],
        out_specs=[
            planar_out,
            planar_out, planar_out, planar_out, planar_out,
        ],
        out_shape=[
            jax.ShapeDtypeStruct((NPAD, NUM_CLASSES), jnp.float32),
            jax.ShapeDtypeStruct((NPAD, NUM_CLASSES), jnp.float32),
            jax.ShapeDtypeStruct((NPAD, NUM_CLASSES), jnp.float32),
            jax.ShapeDtypeStruct((NPAD, NUM_CLASSES), jnp.float32),
            jax.ShapeDtypeStruct((NPAD, NUM_CLASSES), jnp.float32),
        ],
        compiler_params=pltpu.CompilerParams(
            dimension_semantics=("arbitrary",)),
        interpret=_INTERP,
    )(props_pad, ftx, jnp.asarray(_YEXP), wcls_pad, bcls_pad,
      wdx, wdy, wdw, wdh, breg4)


NT = 32
TPT = NPAD // NT
CCAP = 3072
LCAP = 1024
RPG = 1
EMPTY_IDX = 500000.0
INVAL_IDX = 600000.0


def _tocol(row):
    return jnp.reshape(row, (row.shape[1], 1))


def _merge_into(L_ref, chunk):
    allv = jnp.concatenate([L_ref[...], chunk], axis=1)
    sc_row = allv[0:1, :]
    idx_row = allv[1:2, :]
    sc_col = _tocol(sc_row)
    idx_col = _tocol(idx_row)
    rank_col = jnp.zeros((2 * LCAP, 1), jnp.float32)
    for s in range(4):
        scs = sc_row[:, s * 512:(s + 1) * 512]
        idxs = idx_row[:, s * 512:(s + 1) * 512]
        before = ((scs > sc_col) |
                  ((scs == sc_col) & (idxs < idx_col))).astype(jnp.float32)
        rank_col = rank_col + jnp.sum(before, axis=1, keepdims=True)
    lane = jax.lax.broadcasted_iota(jnp.int32, (1, LCAP), 1).astype(jnp.float32)
    w = (rank_col == lane).astype(jnp.float32)
    L_ref[...] = jax.lax.dot_general(allv, w, (((1,), (0,)), ((), ())),
                                     preferred_element_type=jnp.float32)


def _nms_body(cnt_ref, sc_ref, idx_ref, x1_ref, y1_ref, x2_ref, y2_ref,
              dets_ref, num_ref, L_ref, iou_ref,
              vsc_ref, vidx_ref, vx1_ref, vy1_ref, vx2_ref, vy2_ref, sem):
    lane = jax.lax.broadcasted_iota(jnp.int32, (1, LCAP), 1)
    lane_f = lane.astype(jnp.float32)

    L_ref[...] = jnp.concatenate(
        [jnp.zeros((1, LCAP), jnp.float32),
         EMPTY_IDX + lane_f,
         jnp.zeros((6, LCAP), jnp.float32)], axis=0)

    def region_body(wr, carry):
        cntw = cnt_ref[wr, 0]

        @pl.when(cntw > 0)
        def _do_region():
            pairs = [(sc_ref, vsc_ref), (idx_ref, vidx_ref),
                     (x1_ref, vx1_ref), (y1_ref, vy1_ref),
                     (x2_ref, vx2_ref), (y2_ref, vy2_ref)]
            copies = [pltpu.make_async_copy(src.at[pl.ds(wr, 1)], dst, sem)
                      for src, dst in pairs]
            for cp in copies:
                cp.start()
            for cp in copies:
                cp.wait()
            for c in range(CCAP // LCAP):
                @pl.when(cntw > c * LCAP)
                def _do_merge(c=c):
                    rem = cntw - c * LCAP
                    lm = lane < rem
                    raw_sc = vsc_ref[:, pl.ds(c * LCAP, LCAP)]
                    raw_idx = vidx_ref[:, pl.ds(c * LCAP, LCAP)].astype(
                        jnp.float32)
                    csc = jnp.where(lm, raw_sc, -1.0)
                    cidx = jnp.where(lm, raw_idx,
                                     INVAL_IDX + c * LCAP + lane_f)
                    cx1 = jnp.where(lm, vx1_ref[:, pl.ds(c * LCAP, LCAP)], 0.0)
                    cy1 = jnp.where(lm, vy1_ref[:, pl.ds(c * LCAP, LCAP)], 0.0)
                    cx2 = jnp.where(lm, vx2_ref[:, pl.ds(c * LCAP, LCAP)], 0.0)
                    cy2 = jnp.where(lm, vy2_ref[:, pl.ds(c * LCAP, LCAP)], 0.0)
                    chunk = jnp.concatenate(
                        [csc, cidx, cx1, cy1, cx2, cy2,
                         jnp.zeros((2, LCAP), jnp.float32)], axis=0)
                    _merge_into(L_ref, chunk)

        return carry

    jax.lax.fori_loop(0, NT, region_body, jnp.int32(0))

    def _final():
        L = L_ref[...]
        lsc = jnp.where(lane < PRE_NMS, L[0:1, :], 0.0)
        lidx = L[1:2, :]
        x1r = L[2:3, :]
        y1r = L[3:4, :]
        x2r = L[4:5, :]
        y2r = L[5:6, :]
        cls = lidx - jnp.floor(lidx * (1.0 / NUM_CLASSES)) * NUM_CLASSES
        off = cls * 4096.0
        ox1 = x1r + off
        oy1 = y1r + off
        ox2 = x2r + off
        oy2 = y2r + off
        ox1c = _tocol(ox1)
        oy1c = _tocol(oy1)
        ox2c = _tocol(ox2)
        oy2c = _tocol(oy2)
        area_r = jnp.maximum(ox2 - ox1, 0.0) * jnp.maximum(oy2 - oy1, 0.0)
        area_c = jnp.maximum(ox2c - ox1c, 0.0) * jnp.maximum(oy2c - oy1c, 0.0)
        ix1 = jnp.maximum(ox1c, ox1)
        iy1 = jnp.maximum(oy1c, oy1)
        ix2 = jnp.minimum(ox2c, ox2)
        iy2 = jnp.minimum(oy2c, oy2)
        inter = jnp.maximum(ix2 - ix1, 0.0) * jnp.maximum(iy2 - iy1, 0.0)
        iou_ref[...] = inter / (area_c + area_r - inter + 1e-6)

        npos = jnp.sum(jnp.where(lsc > 0.0, 1, 0))

        def body(i, keep):
            row = iou_ref[pl.ds(i, 1), :]
            ki = jnp.sum(jnp.where(lane == i, keep, 0.0))
            sup = (row > IOU_THR) & (lane > i) & (ki > 0.0)
            return jnp.where(sup, 0.0, keep)

        keep0 = jnp.where(lsc > 0.0, 1.0, 0.0)
        kept = jax.lax.fori_loop(0, npos, body, keep0)

        sub2d = jax.lax.broadcasted_iota(jnp.int32, (LCAP, LCAP), 0)
        lane2d = jax.lax.broadcasted_iota(jnp.int32, (LCAP, LCAP), 1)
        m3 = jnp.where(lane2d < sub2d, kept, 0.0)
        pr_col = jnp.sum(m3, axis=1, keepdims=True)
        lane128 = jax.lax.broadcasted_iota(
            jnp.int32, (1, 128), 1).astype(jnp.float32)
        wd = (pr_col == lane128).astype(jnp.float32)
        dmat = jnp.concatenate(
            [lsc, cls, x1r, y1r, x2r, y2r,
             jnp.zeros((2, LCAP), jnp.float32)], axis=0) * kept
        dets_ref[...] = jax.lax.dot_general(
            dmat, wd, (((1,), (0,)), ((), ())),
            preferred_element_type=jnp.float32)
        nk = jnp.sum(kept).astype(jnp.int32)
        num_ref[0, 0] = jnp.minimum(nk, MAX_PER_IMG)

    _final()


def _run_nms(cnt2d, csc, cidx, cx1, cy1, cx2, cy2):
    hbm_spec = pl.BlockSpec(memory_space=pl.ANY)
    return pl.pallas_call(
        _nms_body,
        in_specs=[
            pl.BlockSpec(memory_space=pltpu.SMEM),
            hbm_spec, hbm_spec, hbm_spec, hbm_spec, hbm_spec, hbm_spec,
        ],
        out_specs=[
            pl.BlockSpec((8, 128), lambda: (0, 0)),
            pl.BlockSpec(memory_space=pltpu.SMEM),
        ],
        out_shape=[
            jax.ShapeDtypeStruct((8, 128), jnp.float32),
            jax.ShapeDtypeStruct((1, 1), jnp.int32),
        ],
        scratch_shapes=[
            pltpu.VMEM((8, LCAP), jnp.float32),
            pltpu.VMEM((LCAP, LCAP), jnp.float32),
            pltpu.VMEM((1, CCAP), jnp.float32),
            pltpu.VMEM((1, CCAP), jnp.int32),
            pltpu.VMEM((1, CCAP), jnp.float32),
            pltpu.VMEM((1, CCAP), jnp.float32),
            pltpu.VMEM((1, CCAP), jnp.float32),
            pltpu.VMEM((1, CCAP), jnp.float32),
            pltpu.SemaphoreType.DMA,
        ],
        interpret=_INTERP,
    )(cnt2d, csc, cidx, cx1, cy1, cx2, cy2)


def _sc_compact_body(scores_hbm, bx1_hbm, by1_hbm, bx2_hbm, by2_hbm,
                     cnt_out, sc_out, idx_out, x1_out, y1_out, x2_out, y2_out,
                     sc_v, bx1_v, by1_v, bx2_v, by2_v,
                     csc_v, cidx_v, cx1_v, cy1_v, cx2_v, cy2_v,
                     cnt_v, cnt_s):
    nc = 2
    wid = lax.axis_index("s") * nc + lax.axis_index("c")
    base = wid * TPT
    pltpu.sync_copy(scores_hbm.at[pl.ds(base, TPT)], sc_v)
    pltpu.sync_copy(bx1_hbm.at[pl.ds(base, TPT)], bx1_v)
    pltpu.sync_copy(by1_hbm.at[pl.ds(base, TPT)], by1_v)
    pltpu.sync_copy(bx2_hbm.at[pl.ds(base, TPT)], bx2_v)
    pltpu.sync_copy(by2_hbm.at[pl.ds(base, TPT)], by2_v)
    lane16 = lax.broadcasted_iota(jnp.int32, (16,), 0)
    cnt_s[0] = 0

    def _prefix_inclusive(x):
        for d in (1, 2, 4, 8):
            src = jnp.maximum(lane16 - d, 0)
            y = x.at[src].get(mode="promise_in_bounds")
            x = x + jnp.where(lane16 >= d, y, 0.0)
        return x

    def row_body(r, carry):
        svals = [sc_v[r, pl.ds(v * 16, 16)] for v in range(NUM_CLASSES // 16)]
        masks = [s > SCORE_THR for s in svals]
        many = (masks[0] | masks[1]) | (masks[2] | masks[3]) | masks[4]
        npos_any = plsc.all_reduce_population_count(many)[0]

        @pl.when(npos_any > 0)
        def _process_row():
            cnt = jnp.full((16,), cnt_s[0], jnp.int32)
            for v in range(NUM_CLASSES // 16):
                s = svals[v]
                m = masks[v]
                mf = jnp.where(m, 1.0, 0.0)
                csum = _prefix_inclusive(mf).astype(jnp.int32)
                pos = (cnt + csum) - 1
                gidx = (base + r) * NUM_CLASSES + v * 16 + lane16
                plsc.store_scatter(csc_v, [pos], s, mask=m)
                plsc.store_scatter(cidx_v, [pos], gidx, mask=m)
                plsc.store_scatter(cx1_v, [pos], bx1_v[r, pl.ds(v * 16, 16)],
                                   mask=m)
                plsc.store_scatter(cy1_v, [pos], by1_v[r, pl.ds(v * 16, 16)],
                                   mask=m)
                plsc.store_scatter(cx2_v, [pos], bx2_v[r, pl.ds(v * 16, 16)],
                                   mask=m)
                plsc.store_scatter(cy2_v, [pos], by2_v[r, pl.ds(v * 16, 16)],
                                   mask=m)
                last = csum.at[jnp.full((16,), 15, jnp.int32)].get(
                    mode="promise_in_bounds")
                cnt = cnt + last
            cnt_s[0] = cnt[0]

        return carry

    lax.fori_loop(0, TPT, row_body, jnp.int32(0))
    cnt_v[...] = jnp.full((16,), cnt_s[0], jnp.int32)
    pltpu.sync_copy(cnt_v, cnt_out.at[pl.ds(wid * 16, 16)])
    pltpu.sync_copy(csc_v, sc_out.at[wid])
    pltpu.sync_copy(cidx_v, idx_out.at[wid])
    pltpu.sync_copy(cx1_v, x1_out.at[wid])
    pltpu.sync_copy(cy1_v, y1_out.at[wid])
    pltpu.sync_copy(cx2_v, x2_out.at[wid])
    pltpu.sync_copy(cy2_v, y2_out.at[wid])


def _run_sc_compact(scores_pad, bx1, by1, bx2, by2):
    f32 = jnp.float32
    i32 = jnp.int32
    mesh = plsc.VectorSubcoreMesh(core_axis_name="c", subcore_axis_name="s")
    k = functools.partial(
        pl.kernel,
        mesh=mesh,
        out_type=[
            jax.ShapeDtypeStruct((NT * 16,), i32),
            jax.ShapeDtypeStruct((NT, CCAP), f32),
            jax.ShapeDtypeStruct((NT, CCAP), i32),
            jax.ShapeDtypeStruct((NT, CCAP), f32),
            jax.ShapeDtypeStruct((NT, CCAP), f32),
            jax.ShapeDtypeStruct((NT, CCAP), f32),
            jax.ShapeDtypeStruct((NT, CCAP), f32),
        ],
        scratch_types=[
            pltpu.VMEM((TPT, NUM_CLASSES), f32),
            pltpu.VMEM((TPT, NUM_CLASSES), f32),
            pltpu.VMEM((TPT, NUM_CLASSES), f32),
            pltpu.VMEM((TPT, NUM_CLASSES), f32),
            pltpu.VMEM((TPT, NUM_CLASSES), f32),
            pltpu.VMEM((CCAP,), f32),
            pltpu.VMEM((CCAP,), i32),
            pltpu.VMEM((CCAP,), f32),
            pltpu.VMEM((CCAP,), f32),
            pltpu.VMEM((CCAP,), f32),
            pltpu.VMEM((CCAP,), f32),
            pltpu.VMEM((16,), i32),
            pltpu.SMEM((1,), i32),
        ],
        compiler_params=pltpu.CompilerParams(needs_layout_passes=False),
    )(_sc_compact_body)
    return k(scores_pad, bx1, by1, bx2, by2)


def _compact_emul_jax(scores_pad, bx1, by1, bx2, by2):
    sc3 = scores_pad.reshape(NT, TPT * NUM_CLASSES)
    m = sc3 > SCORE_THR
    cnt = jnp.sum(m.astype(jnp.int32), axis=1)
    order = jnp.argsort(~m, axis=1, stable=True)[:, :CCAP]
    csc = jnp.take_along_axis(sc3, order, axis=1)
    base = (jnp.arange(NT, dtype=jnp.int32) * TPT * NUM_CLASSES)[:, None]
    cidx = base + order.astype(jnp.int32)
    outs = [jnp.take_along_axis(b.reshape(NT, TPT * NUM_CLASSES), order, axis=1)
            for b in (bx1, by1, bx2, by2)]
    return (cnt, csc, cidx, *outs)


def kernel(feat, proposals, W_cls, b_cls, W_reg, b_reg):
    ftx = jnp.transpose(feat[0], (2, 1, 0)).reshape(W, H * C)
    props_pad = jnp.pad(proposals[0], ((0, NPAD - N), (0, 0)))
    wcls_pad = jnp.pad(W_cls, ((0, 0), (0, 128 - (NUM_CLASSES + 1))))
    bcls_pad = jnp.pad(b_cls, (0, 128 - (NUM_CLASSES + 1))).reshape(1, 128)
    breg4 = jnp.transpose(b_reg.reshape(NUM_CLASSES, 4))
    wdx = W_reg[:, 0::4]
    wdy = W_reg[:, 1::4]
    wdw = W_reg[:, 2::4]
    wdh = W_reg[:, 3::4]

    scores_pad, bx1, by1, bx2, by2 = _run_head(
        props_pad, ftx, wcls_pad, bcls_pad, wdx, wdy, wdw, wdh, breg4)
    if _USE_SC:
        cntv, csc, cidx, cx1, cy1, cx2, cy2 = _run_sc_compact(
            scores_pad, bx1, by1, bx2, by2)
        cnt2d = cntv.reshape(NT, 16)
    else:
        cnt, csc, cidx, cx1, cy1, cx2, cy2 = _compact_emul_jax(
            scores_pad, bx1, by1, bx2, by2)
        cnt2d = jnp.broadcast_to(cnt[:, None], (NT, 16))
    dets, num = _run_nms(cnt2d, csc, cidx, cx1, cy1, cx2, cy2)
    det_sc = dets[0, :MAX_PER_IMG]
    det_cls = jnp.where(det_sc > 0.0,
                        dets[1, :MAX_PER_IMG].astype(jnp.int32), -1)
    det_bx = jnp.transpose(dets[2:6, :MAX_PER_IMG])
    num_s = num[0, 0]
    return (num_s[None], det_bx[None], det_sc[None], det_cls[None])
